# Initial kernel scaffold; baseline (speedup 1.0000x reference)
import jax
import jax.numpy as jnp
from jax.experimental import pallas as pl


def kernel(pos, batch, f_in, edge_index, W1, W2, W3, gamma_s, beta_s, gamma_v):
    raise NotImplementedError("write your pallas kernel here")


# speedup vs baseline: 4.5482x; 4.5482x over previous
import functools

import jax
import jax.numpy as jnp
import numpy as np
from jax import lax
from jax.experimental import pallas as pl
from jax.experimental.pallas import tpu as pltpu
from jax.experimental.pallas import tpu_sc as plsc

N_NODES = 50000
N_EDGES = 800000
RADIUS = 5.0
NBASIS = 20

NC, NS = 2, 16
NW = NC * NS
CB = 128
CPW = 196
E_PAD = NW * CPW * CB
CPT = E_PAD // NC // NS // CB
ACC_ROWS = 50048
RPT = ACC_ROWS // NS
DUMP_ROW = N_NODES

BE = 512

_SQ2 = float(np.sqrt(2.0))
_SQ3 = float(np.sqrt(3.0))
_ALPHA = float(1.0 / np.sqrt(12.0))
_EMBC = float(1.14136 * np.exp(2.0) * np.sqrt(float(NBASIS)))
_STEP = float(RADIUS / (NBASIS + 1))

@functools.cache
def _gather_phase():
    mesh = plsc.VectorSubcoreMesh(core_axis_name="c", subcore_axis_name="s")
    return functools.partial(
        pl.kernel,
        out_type=(
            jax.ShapeDtypeStruct((E_PAD, 32), jnp.float32),
            jax.ShapeDtypeStruct((E_PAD, 16), jnp.float32),
        ),
        mesh=mesh,
        scratch_types=[
            pltpu.VMEM((CB,), jnp.int32),
            pltpu.VMEM((CB,), jnp.int32),
            pltpu.VMEM((CB, 32), jnp.float32),
            pltpu.VMEM((CB, 16), jnp.float32),
            pltpu.SemaphoreType.DMA,
            pltpu.SemaphoreType.DMA,
        ],
        compiler_params=pltpu.CompilerParams(use_tc_tiling_on_sc=False),
    )(_gather_body)


def _gather_body(src_hbm, dst_hbm, tab32_hbm, tab16_hbm, o32_hbm, o16_hbm,
                 idx_s, idx_d, buf32, buf16, sem_a, sem_b):
    wid = lax.axis_index("s") * NC + lax.axis_index("c")

    def body(ci, _):
        base = (wid * CPW + ci) * CB
        pltpu.sync_copy(src_hbm.at[pl.ds(base, CB)], idx_s)
        pltpu.sync_copy(dst_hbm.at[pl.ds(base, CB)], idx_d)
        a = pltpu.async_copy(tab32_hbm.at[idx_s], buf32, sem_a)
        b = pltpu.async_copy(tab16_hbm.at[idx_d], buf16, sem_b)
        a.wait()
        b.wait()
        pltpu.sync_copy(buf32, o32_hbm.at[pl.ds(base, CB)])
        pltpu.sync_copy(buf16, o16_hbm.at[pl.ds(base, CB)])
        return 0

    lax.fori_loop(0, CPW, body, 0)


@functools.cache
def _scatter_phase():
    mesh = plsc.VectorSubcoreMesh(core_axis_name="c", subcore_axis_name="s")
    return functools.partial(
        pl.kernel,
        out_type=jax.ShapeDtypeStruct((NC * ACC_ROWS, 24), jnp.float32),
        mesh=mesh,
        scratch_types=[
            pltpu.VMEM((CB,), jnp.int32),
            pltpu.VMEM((CB, 24), jnp.float32),
            pltpu.VMEM_SHARED((ACC_ROWS, 24), jnp.float32),
        ],
        compiler_params=pltpu.CompilerParams(use_tc_tiling_on_sc=False),
    )(_scatter_body)


def _scatter_body(dst_hbm, fe_hbm, zeros_hbm, out_hbm, idx_v, buf, acc):
    cid = lax.axis_index("c")
    sid = lax.axis_index("s")

    pltpu.sync_copy(zeros_hbm.at[pl.ds(sid * RPT, RPT)],
                    acc.at[pl.ds(sid * RPT, RPT)])
    plsc.subcore_barrier()

    half = E_PAD // NC

    def body(ci, _):
        base = cid * half + (sid * CPT + ci) * CB
        pltpu.sync_copy(dst_hbm.at[pl.ds(base, CB)], idx_v)
        pltpu.sync_copy(fe_hbm.at[pl.ds(base, CB)], buf)
        pltpu.sync_copy(buf, acc.at[idx_v], add=True)
        return 0

    lax.fori_loop(0, CPT, body, 0)

    plsc.subcore_barrier()
    pltpu.sync_copy(acc.at[pl.ds(sid * RPT, RPT)],
                    out_hbm.at[pl.ds(cid * ACC_ROWS + sid * RPT, RPT)])


def _dense_body(x32_ref, x16_ref, w1t_ref, w2t_ref, w3t_ref, out_ref):
    xt = x32_ref[...].T
    pdt = x16_ref[...].T
    vec = pdt[0:3] - xt[0:3]
    r2 = vec[0:1] * vec[0:1] + vec[1:2] * vec[1:2] + vec[2:3] * vec[2:3]
    r = jnp.sqrt(r2 + 1e-12)
    y1 = _SQ3 * vec / r

    vals = _STEP * (1.0 + lax.broadcasted_iota(
        jnp.int32, (NBASIS, 1), 0).astype(jnp.float32))
    d = (r - vals) * (1.0 / _STEP)
    u = 1.0 - d * d
    good = u > 0.0
    emb = jnp.where(good, _EMBC * jnp.exp(-2.0 / jnp.where(good, u, 1.0)), 0.0)

    h = jnp.maximum(w1t_ref[...] @ emb, 0.0) * _SQ2
    h = jnp.maximum(w2t_ref[...] @ h, 0.0) * _SQ2
    w = w3t_ref[...] @ h

    s = xt[3:11]
    v = xt[11:23]

    dots = []
    for k in range(4):
        dk = (v[3 * k:3 * k + 1] * y1[0:1]
              + v[3 * k + 1:3 * k + 2] * y1[1:2]
              + v[3 * k + 2:3 * k + 3] * y1[2:3]) * (1.0 / _SQ3)
        dots.append(dk)

    out0 = s[0:1] * w[0:8]
    for i in range(1, 8):
        out0 = out0 + s[i:i + 1] * w[8 * i:8 * i + 8]
    for k in range(4):
        out0 = out0 + dots[k] * w[64 + 8 * k:72 + 8 * k]
    out0 = out0 * _ALPHA

    p = s[0:1] * w[96:100]
    for i in range(1, 8):
        p = p + s[i:i + 1] * w[96 + 4 * i:100 + 4 * i]
    q = []
    for c in range(3):
        qc = v[c:c + 1] * w[128:132]
        for k in range(1, 4):
            qc = qc + v[3 * k + c:3 * k + c + 1] * w[128 + 4 * k:132 + 4 * k]
        q.append(qc)

    rows = [out0]
    for o in range(4):
        for c in range(3):
            rows.append((p[o:o + 1] * y1[c:c + 1] + q[c][o:o + 1]) * _ALPHA)
    rows.append(jnp.zeros((4, BE), jnp.float32))
    fe = jnp.concatenate(rows, axis=0)
    out_ref[...] = fe.T


def _dense_phase(x32, x16, w1t, w2t, w3t):
    grid = (E_PAD // BE,)
    return pl.pallas_call(
        _dense_body,
        grid=grid,
        in_specs=[
            pl.BlockSpec((BE, 32), lambda i: (i, 0)),
            pl.BlockSpec((BE, 16), lambda i: (i, 0)),
            pl.BlockSpec((20, 20), lambda i: (0, 0)),
            pl.BlockSpec((20, 20), lambda i: (0, 0)),
            pl.BlockSpec((144, 20), lambda i: (0, 0)),
        ],
        out_specs=pl.BlockSpec((BE, 24), lambda i: (i, 0)),
        out_shape=jax.ShapeDtypeStruct((E_PAD, 24), jnp.float32),
    )(x32, x16, w1t, w2t, w3t)


BN_BLK = 3128
BN_NBLK = ACC_ROWS // BN_BLK


def _stats_body(pa_ref, pb_ref, out_ref, acc_ref):
    i = pl.program_id(0)

    @pl.when(i == 0)
    def _():
        acc_ref[...] = jnp.zeros_like(acc_ref)

    f = pa_ref[...] + pb_ref[...]
    rows = i * BN_BLK + lax.broadcasted_iota(jnp.int32, (BN_BLK, 24), 0)
    fm = jnp.where(rows < N_NODES, f, 0.0)
    acc_ref[0:1] += jnp.sum(fm, axis=0, keepdims=True)
    acc_ref[1:2] += jnp.sum(fm * fm, axis=0, keepdims=True)

    @pl.when(i == BN_NBLK - 1)
    def _():
        out_ref[...] = acc_ref[...]


def _norm_body(pa_ref, pb_ref, st_ref, grow_ref, brow_ref, out_ref):
    f = pa_ref[...] + pb_ref[...]
    inv_n = 1.0 / float(N_NODES)
    mu = st_ref[0:1] * inv_n
    sq = st_ref[1:2] * inv_n
    var = sq - mu * mu
    lane = lax.broadcasted_iota(jnp.int32, (24, 24), 0)
    lane_t = lax.broadcasted_iota(jnp.int32, (24, 24), 1)
    vlane = (lane >= 8) & (lane < 20) & (lane_t >= 8) & (lane_t < 20)
    m3 = jnp.where(vlane & ((lane - 8) // 3 == (lane_t - 8) // 3), 1.0, 0.0)
    n2 = sq @ m3
    s_lane = lax.broadcasted_iota(jnp.int32, (1, 24), 1) < 8
    denom = jnp.sqrt(jnp.where(s_lane, var, n2) + 1e-5)
    norm = jnp.where(s_lane, f - mu, f) / denom
    res = norm * grow_ref[...] + brow_ref[...]
    out_ref[...] = res[:, 0:20]


def _bn_phase(parts, grow, brow):
    pa_spec = pl.BlockSpec((BN_BLK, 24), lambda i: (i, 0))
    pb_spec = pl.BlockSpec((BN_BLK, 24), lambda i: (i + BN_NBLK, 0))
    stats = pl.pallas_call(
        _stats_body,
        grid=(BN_NBLK,),
        in_specs=[pa_spec, pb_spec],
        out_specs=pl.BlockSpec((2, 24), lambda i: (0, 0)),
        out_shape=jax.ShapeDtypeStruct((2, 24), jnp.float32),
        scratch_shapes=[pltpu.VMEM((2, 24), jnp.float32)],
    )(parts, parts)
    return pl.pallas_call(
        _norm_body,
        grid=(BN_NBLK,),
        in_specs=[
            pa_spec,
            pb_spec,
            pl.BlockSpec((2, 24), lambda i: (0, 0)),
            pl.BlockSpec((1, 24), lambda i: (0, 0)),
            pl.BlockSpec((1, 24), lambda i: (0, 0)),
        ],
        out_specs=pl.BlockSpec((BN_BLK, 20), lambda i: (i, 0)),
        out_shape=jax.ShapeDtypeStruct((N_NODES, 20), jnp.float32),
    )(parts, parts, stats, grow, brow)


def kernel(pos, batch, f_in, edge_index, W1, W2, W3, gamma_s, beta_s, gamma_v):
    src = edge_index[0]
    dst = edge_index[1]
    pad = E_PAD - N_EDGES
    src_p = jnp.concatenate([src, jnp.zeros((pad,), jnp.int32)])
    dst_p = jnp.concatenate([dst, jnp.full((pad,), DUMP_ROW, jnp.int32)])

    tab32 = jnp.concatenate(
        [pos, f_in, jnp.zeros((N_NODES, 9), jnp.float32)], axis=1)
    tab16 = jnp.concatenate(
        [pos, jnp.zeros((N_NODES, 13), jnp.float32)], axis=1)

    x32, x16 = _gather_phase()(src_p, dst_p, tab32, tab16)

    w1t = (W1 * (1.0 / np.sqrt(float(NBASIS)))).T
    w2t = (W2 * (1.0 / np.sqrt(20.0))).T
    w3t = (W3 * (1.0 / np.sqrt(20.0))).T
    fe = _dense_phase(x32, x16, w1t, w2t, w3t)

    zeros_acc = jnp.zeros((ACC_ROWS, 24), jnp.float32)
    parts = _scatter_phase()(dst_p, fe, zeros_acc)

    grow = jnp.concatenate(
        [gamma_s, jnp.repeat(gamma_v, 3), jnp.zeros((4,), jnp.float32)]
    ).reshape(1, 24)
    brow = jnp.concatenate(
        [beta_s, jnp.zeros((16,), jnp.float32)]).reshape(1, 24)
    return _bn_phase(parts, grow, brow)


# speedup vs baseline: 6.2075x; 1.3648x over previous
import functools

import jax
import jax.numpy as jnp
import numpy as np
from jax import lax
from jax.experimental import pallas as pl
from jax.experimental.pallas import tpu as pltpu
from jax.experimental.pallas import tpu_sc as plsc

N_NODES = 50000
N_EDGES = 800000
RADIUS = 5.0
NBASIS = 20

NC, NS = 2, 16
NW = NC * NS
CB = 128
CPW = 196
E_PAD = NW * CPW * CB
CPT = E_PAD // NC // NS // CB
ACC_ROWS = 50048
RPT = ACC_ROWS // NS
DUMP_ROW = N_NODES

BE = 2048

_SQ2 = float(np.sqrt(2.0))
_SQ3 = float(np.sqrt(3.0))
_ALPHA = float(1.0 / np.sqrt(12.0))
_EMBC = float(1.14136 * np.exp(2.0) * np.sqrt(float(NBASIS)))
_STEP = float(RADIUS / (NBASIS + 1))

@functools.cache
def _gather_phase():
    mesh = plsc.VectorSubcoreMesh(core_axis_name="c", subcore_axis_name="s")
    return functools.partial(
        pl.kernel,
        out_type=(
            jax.ShapeDtypeStruct((E_PAD, 32), jnp.float32),
            jax.ShapeDtypeStruct((E_PAD, 16), jnp.float32),
        ),
        mesh=mesh,
        scratch_types=[
            pltpu.VMEM((CB,), jnp.int32),
            pltpu.VMEM((CB,), jnp.int32),
            pltpu.VMEM((CB, 32), jnp.float32),
            pltpu.VMEM((CB, 16), jnp.float32),
            pltpu.SemaphoreType.DMA,
            pltpu.SemaphoreType.DMA,
        ],
        compiler_params=pltpu.CompilerParams(use_tc_tiling_on_sc=False),
    )(_gather_body)


def _gather_body(src_hbm, dst_hbm, tab32_hbm, tab16_hbm, o32_hbm, o16_hbm,
                 idx_s, idx_d, buf32, buf16, sem_a, sem_b):
    wid = lax.axis_index("s") * NC + lax.axis_index("c")

    def body(ci, _):
        base = (wid * CPW + ci) * CB
        pltpu.sync_copy(src_hbm.at[pl.ds(base, CB)], idx_s)
        pltpu.sync_copy(dst_hbm.at[pl.ds(base, CB)], idx_d)
        a = pltpu.async_copy(tab32_hbm.at[idx_s], buf32, sem_a)
        b = pltpu.async_copy(tab16_hbm.at[idx_d], buf16, sem_b)
        a.wait()
        b.wait()
        pltpu.sync_copy(buf32, o32_hbm.at[pl.ds(base, CB)])
        pltpu.sync_copy(buf16, o16_hbm.at[pl.ds(base, CB)])
        return 0

    lax.fori_loop(0, CPW, body, 0)


@functools.cache
def _scatter_phase():
    mesh = plsc.VectorSubcoreMesh(core_axis_name="c", subcore_axis_name="s")
    return functools.partial(
        pl.kernel,
        out_type=jax.ShapeDtypeStruct((NC * ACC_ROWS, 24), jnp.float32),
        mesh=mesh,
        scratch_types=[
            pltpu.VMEM((CB,), jnp.int32),
            pltpu.VMEM((CB, 24), jnp.float32),
            pltpu.VMEM_SHARED((ACC_ROWS, 24), jnp.float32),
        ],
        compiler_params=pltpu.CompilerParams(use_tc_tiling_on_sc=False),
    )(_scatter_body)


def _scatter_body(dst_hbm, fe_hbm, zeros_hbm, out_hbm, idx_v, buf, acc):
    cid = lax.axis_index("c")
    sid = lax.axis_index("s")

    pltpu.sync_copy(zeros_hbm.at[pl.ds(sid * RPT, RPT)],
                    acc.at[pl.ds(sid * RPT, RPT)])
    plsc.subcore_barrier()

    half = E_PAD // NC

    def body(ci, _):
        base = cid * half + (sid * CPT + ci) * CB
        pltpu.sync_copy(dst_hbm.at[pl.ds(base, CB)], idx_v)
        pltpu.sync_copy(fe_hbm.at[pl.ds(base, CB)], buf)
        pltpu.sync_copy(buf, acc.at[idx_v], add=True)
        return 0

    lax.fori_loop(0, CPT, body, 0)

    plsc.subcore_barrier()
    pltpu.sync_copy(acc.at[pl.ds(sid * RPT, RPT)],
                    out_hbm.at[pl.ds(cid * ACC_ROWS + sid * RPT, RPT)])


def _dense_body(x32_ref, x16_ref, w1t_ref, w2t_ref, w3t_ref, out_ref):
    xt = x32_ref[...].T
    pdt = x16_ref[...].T
    vec = pdt[0:3] - xt[0:3]
    r2 = vec[0:1] * vec[0:1] + vec[1:2] * vec[1:2] + vec[2:3] * vec[2:3] + 1e-12
    rinv = lax.rsqrt(r2)
    r = r2 * rinv
    y1 = _SQ3 * vec * rinv

    vals = _STEP * (1.0 + lax.broadcasted_iota(
        jnp.int32, (NBASIS, 1), 0).astype(jnp.float32))
    d = (r - vals) * (1.0 / _STEP)
    u = 1.0 - d * d
    good = u > 0.0
    emb = jnp.where(good, _EMBC * jnp.exp(-2.0 / jnp.where(good, u, 1.0)), 0.0)

    f32 = jnp.float32
    h = jnp.dot(w1t_ref[...], emb.astype(jnp.bfloat16),
                preferred_element_type=f32)
    h = (jnp.maximum(h, 0.0) * _SQ2).astype(jnp.bfloat16)
    h = jnp.dot(w2t_ref[...], h, preferred_element_type=f32)
    h = (jnp.maximum(h, 0.0) * _SQ2).astype(jnp.bfloat16)
    w = jnp.dot(w3t_ref[...], h, preferred_element_type=f32)

    s = xt[3:11]
    v = xt[11:23]

    dots = []
    for k in range(4):
        dk = (v[3 * k:3 * k + 1] * y1[0:1]
              + v[3 * k + 1:3 * k + 2] * y1[1:2]
              + v[3 * k + 2:3 * k + 3] * y1[2:3]) * (1.0 / _SQ3)
        dots.append(dk)

    out0 = s[0:1] * w[0:8]
    for i in range(1, 8):
        out0 = out0 + s[i:i + 1] * w[8 * i:8 * i + 8]
    for k in range(4):
        out0 = out0 + dots[k] * w[64 + 8 * k:72 + 8 * k]
    out0 = out0 * _ALPHA

    p = s[0:1] * w[96:100]
    for i in range(1, 8):
        p = p + s[i:i + 1] * w[96 + 4 * i:100 + 4 * i]
    q = []
    for c in range(3):
        qc = v[c:c + 1] * w[128:132]
        for k in range(1, 4):
            qc = qc + v[3 * k + c:3 * k + c + 1] * w[128 + 4 * k:132 + 4 * k]
        q.append(qc)

    rows = [out0]
    for c in range(3):
        rows.append((p * y1[c:c + 1] + q[c]) * _ALPHA)
    rows.append(jnp.zeros((4, BE), jnp.float32))
    fe = jnp.concatenate(rows, axis=0)
    out_ref[...] = fe.T


def _dense_phase(x32, x16, w1t, w2t, w3t):
    grid = (E_PAD // BE,)
    return pl.pallas_call(
        _dense_body,
        grid=grid,
        in_specs=[
            pl.BlockSpec((BE, 32), lambda i: (i, 0)),
            pl.BlockSpec((BE, 16), lambda i: (i, 0)),
            pl.BlockSpec((20, 20), lambda i: (0, 0)),
            pl.BlockSpec((20, 20), lambda i: (0, 0)),
            pl.BlockSpec((144, 20), lambda i: (0, 0)),
        ],
        out_specs=pl.BlockSpec((BE, 24), lambda i: (i, 0)),
        out_shape=jax.ShapeDtypeStruct((E_PAD, 24), jnp.float32),
    )(x32, x16, w1t, w2t, w3t)


BN_BLK = 3128
BN_NBLK = ACC_ROWS // BN_BLK


def _stats_body(pa_ref, pb_ref, out_ref, acc_ref):
    i = pl.program_id(0)

    @pl.when(i == 0)
    def _():
        acc_ref[...] = jnp.zeros_like(acc_ref)

    f = pa_ref[...] + pb_ref[...]
    rows = i * BN_BLK + lax.broadcasted_iota(jnp.int32, (BN_BLK, 24), 0)
    fm = jnp.where(rows < N_NODES, f, 0.0)
    acc_ref[0:1] += jnp.sum(fm, axis=0, keepdims=True)
    acc_ref[1:2] += jnp.sum(fm * fm, axis=0, keepdims=True)

    @pl.when(i == BN_NBLK - 1)
    def _():
        out_ref[...] = acc_ref[...]


def _norm_body(pa_ref, pb_ref, st_ref, grow_ref, brow_ref, out_ref):
    f = pa_ref[...] + pb_ref[...]
    inv_n = 1.0 / float(N_NODES)
    mu = st_ref[0:1] * inv_n
    sq = st_ref[1:2] * inv_n
    var = sq - mu * mu
    lane = lax.broadcasted_iota(jnp.int32, (24, 24), 0)
    lane_t = lax.broadcasted_iota(jnp.int32, (24, 24), 1)
    vlane = (lane >= 8) & (lane < 20) & (lane_t >= 8) & (lane_t < 20)
    m3 = jnp.where(vlane & ((lane - 8) % 4 == (lane_t - 8) % 4), 1.0, 0.0)
    n2 = sq @ m3
    s_lane = lax.broadcasted_iota(jnp.int32, (1, 24), 1) < 8
    denom = jnp.sqrt(jnp.where(s_lane, var, n2) + 1e-5)
    norm = jnp.where(s_lane, f - mu, f) / denom
    res = norm * grow_ref[...] + brow_ref[...]
    sblock = (lane == lane_t) & (lane_t < 8)
    vperm = vlane & (lane - 8 == ((lane_t - 8) % 3) * 4 + (lane_t - 8) // 3)
    pmat = jnp.where(sblock | vperm, 1.0, 0.0)
    res = res @ pmat
    out_ref[...] = res[:, 0:20]


def _bn_phase(parts, grow, brow):
    pa_spec = pl.BlockSpec((BN_BLK, 24), lambda i: (i, 0))
    pb_spec = pl.BlockSpec((BN_BLK, 24), lambda i: (i + BN_NBLK, 0))
    stats = pl.pallas_call(
        _stats_body,
        grid=(BN_NBLK,),
        in_specs=[pa_spec, pb_spec],
        out_specs=pl.BlockSpec((2, 24), lambda i: (0, 0)),
        out_shape=jax.ShapeDtypeStruct((2, 24), jnp.float32),
        scratch_shapes=[pltpu.VMEM((2, 24), jnp.float32)],
    )(parts, parts)
    return pl.pallas_call(
        _norm_body,
        grid=(BN_NBLK,),
        in_specs=[
            pa_spec,
            pb_spec,
            pl.BlockSpec((2, 24), lambda i: (0, 0)),
            pl.BlockSpec((1, 24), lambda i: (0, 0)),
            pl.BlockSpec((1, 24), lambda i: (0, 0)),
        ],
        out_specs=pl.BlockSpec((BN_BLK, 20), lambda i: (i, 0)),
        out_shape=jax.ShapeDtypeStruct((N_NODES, 20), jnp.float32),
    )(parts, parts, stats, grow, brow)


def kernel(pos, batch, f_in, edge_index, W1, W2, W3, gamma_s, beta_s, gamma_v):
    src = edge_index[0]
    dst = edge_index[1]
    pad = E_PAD - N_EDGES
    src_p = jnp.concatenate([src, jnp.zeros((pad,), jnp.int32)])
    dst_p = jnp.concatenate([dst, jnp.full((pad,), DUMP_ROW, jnp.int32)])

    tab32 = jnp.concatenate(
        [pos, f_in, jnp.zeros((N_NODES, 9), jnp.float32)], axis=1)
    tab16 = jnp.concatenate(
        [pos, jnp.zeros((N_NODES, 13), jnp.float32)], axis=1)

    x32, x16 = _gather_phase()(src_p, dst_p, tab32, tab16)

    w1t = (W1 * (1.0 / np.sqrt(float(NBASIS)))).T.astype(jnp.bfloat16)
    w2t = (W2 * (1.0 / np.sqrt(20.0))).T.astype(jnp.bfloat16)
    w3t = (W3 * (1.0 / np.sqrt(20.0))).T.astype(jnp.bfloat16)
    fe = _dense_phase(x32, x16, w1t, w2t, w3t)

    zeros_acc = jnp.zeros((ACC_ROWS, 24), jnp.float32)
    parts = _scatter_phase()(dst_p, fe, zeros_acc)

    grow = jnp.concatenate(
        [gamma_s, jnp.tile(gamma_v, 3), jnp.zeros((4,), jnp.float32)]
    ).reshape(1, 24)
    brow = jnp.concatenate(
        [beta_s, jnp.zeros((16,), jnp.float32)]).reshape(1, 24)
    return _bn_phase(parts, grow, brow)


# speedup vs baseline: 10.1777x; 1.6396x over previous
import functools

import jax
import jax.numpy as jnp
import numpy as np
from jax import lax
from jax.experimental import pallas as pl
from jax.experimental.pallas import tpu as pltpu
from jax.experimental.pallas import tpu_sc as plsc

N_NODES = 50000
N_EDGES = 800000
RADIUS = 5.0
NBASIS = 20

NC, NS = 2, 16
NW = NC * NS
CB = 128
CPW = 196
E_PAD = NW * CPW * CB
CPT = E_PAD // NC // NS // CB
ACC_ROWS = 50048
RPT = ACC_ROWS // NS
DUMP_ROW = N_NODES

BE = 2048

_SQ2 = float(np.sqrt(2.0))
_SQ3 = float(np.sqrt(3.0))
_ALPHA = float(1.0 / np.sqrt(12.0))
_EMBC = float(1.14136 * np.exp(2.0) * np.sqrt(float(NBASIS)))
_STEP = float(RADIUS / (NBASIS + 1))

@functools.cache
def _gather_phase():
    mesh = plsc.VectorSubcoreMesh(core_axis_name="c", subcore_axis_name="s")
    return functools.partial(
        pl.kernel,
        out_type=(
            jax.ShapeDtypeStruct((E_PAD, 32), jnp.float32),
            jax.ShapeDtypeStruct((E_PAD, 32), jnp.float32),
        ),
        mesh=mesh,
        scratch_types=[
            pltpu.VMEM((CB,), jnp.int32),
            pltpu.VMEM((CB,), jnp.int32),
            pltpu.VMEM((CB, 32), jnp.float32),
            pltpu.VMEM((CB, 32), jnp.float32),
            pltpu.SemaphoreType.DMA,
            pltpu.SemaphoreType.DMA,
        ],
        compiler_params=pltpu.CompilerParams(use_tc_tiling_on_sc=False),
    )(_gather_body)


def _gather_body(src_hbm, dst_hbm, tab32_hbm, os_hbm, od_hbm,
                 idx_s, idx_d, buf_s, buf_d, sem_a, sem_b):
    wid = lax.axis_index("s") * NC + lax.axis_index("c")

    def body(ci, _):
        base = (wid * CPW + ci) * CB
        pltpu.sync_copy(src_hbm.at[pl.ds(base, CB)], idx_s)
        pltpu.sync_copy(dst_hbm.at[pl.ds(base, CB)], idx_d)
        a = pltpu.async_copy(tab32_hbm.at[idx_s], buf_s, sem_a)
        b = pltpu.async_copy(tab32_hbm.at[idx_d], buf_d, sem_b)
        a.wait()
        b.wait()
        pltpu.sync_copy(buf_s, os_hbm.at[pl.ds(base, CB)])
        pltpu.sync_copy(buf_d, od_hbm.at[pl.ds(base, CB)])
        return 0

    lax.fori_loop(0, CPW, body, 0)


@functools.cache
def _scatter_phase():
    mesh = plsc.VectorSubcoreMesh(core_axis_name="c", subcore_axis_name="s")
    return functools.partial(
        pl.kernel,
        out_type=jax.ShapeDtypeStruct((NC * ACC_ROWS, 32), jnp.float32),
        mesh=mesh,
        scratch_types=[
            pltpu.VMEM((CB,), jnp.int32),
            pltpu.VMEM((CB, 32), jnp.float32),
            pltpu.VMEM_SHARED((ACC_ROWS, 32), jnp.float32),
        ],
        compiler_params=pltpu.CompilerParams(use_tc_tiling_on_sc=False),
    )(_scatter_body)


def _scatter_body(dst_hbm, fe_hbm, zeros_hbm, out_hbm, idx_v, buf, acc):
    cid = lax.axis_index("c")
    sid = lax.axis_index("s")

    pltpu.sync_copy(zeros_hbm.at[pl.ds(sid * RPT, RPT)],
                    acc.at[pl.ds(sid * RPT, RPT)])
    plsc.subcore_barrier()

    half = E_PAD // NC

    def body(ci, _):
        base = cid * half + (sid * CPT + ci) * CB
        pltpu.sync_copy(dst_hbm.at[pl.ds(base, CB)], idx_v)
        pltpu.sync_copy(fe_hbm.at[pl.ds(base, CB)], buf)
        pltpu.sync_copy(buf, acc.at[idx_v], add=True)
        return 0

    lax.fori_loop(0, CPT, body, 0)

    plsc.subcore_barrier()
    pltpu.sync_copy(acc.at[pl.ds(sid * RPT, RPT)],
                    out_hbm.at[pl.ds(cid * ACC_ROWS + sid * RPT, RPT)])


def _dense_body(xs_ref, xd_ref, w1t_ref, w2t_ref, w3t_ref, out_ref):
    BE4 = BE // 4
    y = xs_ref[...].T
    xt = jnp.concatenate([y[32 * r:32 * r + 32] for r in range(4)], axis=1)
    z = xd_ref[...].T
    pdt = jnp.concatenate([z[32 * r:32 * r + 3] for r in range(4)], axis=1)
    vec = pdt - xt[0:3]
    r2 = vec[0:1] * vec[0:1] + vec[1:2] * vec[1:2] + vec[2:3] * vec[2:3] + 1e-12
    rinv = lax.rsqrt(r2)
    r = r2 * rinv
    y1 = _SQ3 * vec * rinv

    vals = _STEP * (1.0 + lax.broadcasted_iota(
        jnp.int32, (NBASIS, 1), 0).astype(jnp.float32))
    d = (r - vals) * (1.0 / _STEP)
    u = 1.0 - d * d
    good = u > 0.0
    emb = jnp.where(good, _EMBC * jnp.exp(-2.0 / jnp.where(good, u, 1.0)), 0.0)

    f32 = jnp.float32
    h = jnp.dot(w1t_ref[...], emb.astype(jnp.bfloat16),
                preferred_element_type=f32)
    h = (jnp.maximum(h, 0.0) * _SQ2).astype(jnp.bfloat16)
    h = jnp.dot(w2t_ref[...], h, preferred_element_type=f32)
    h = (jnp.maximum(h, 0.0) * _SQ2).astype(jnp.bfloat16)
    w = jnp.dot(w3t_ref[...], h, preferred_element_type=f32)

    s = xt[3:11]
    v = xt[11:23]

    dots = []
    for k in range(4):
        dk = (v[3 * k:3 * k + 1] * y1[0:1]
              + v[3 * k + 1:3 * k + 2] * y1[1:2]
              + v[3 * k + 2:3 * k + 3] * y1[2:3]) * (1.0 / _SQ3)
        dots.append(dk)

    out0 = s[0:1] * w[0:8]
    for i in range(1, 8):
        out0 = out0 + s[i:i + 1] * w[8 * i:8 * i + 8]
    for k in range(4):
        out0 = out0 + dots[k] * w[64 + 8 * k:72 + 8 * k]
    out0 = out0 * _ALPHA

    p = s[0:1] * w[96:100]
    for i in range(1, 8):
        p = p + s[i:i + 1] * w[96 + 4 * i:100 + 4 * i]
    q = []
    for c in range(3):
        qc = v[c:c + 1] * w[128:132]
        for k in range(1, 4):
            qc = qc + v[3 * k + c:3 * k + c + 1] * w[128 + 4 * k:132 + 4 * k]
        q.append(qc)

    rows = [out0]
    for c in range(3):
        rows.append((p * y1[c:c + 1] + q[c]) * _ALPHA)
    rows.append(jnp.zeros((12, BE), jnp.float32))
    fe = jnp.concatenate(rows, axis=0)
    ft = fe.T
    out_ref[...] = jnp.concatenate(
        [ft[r * BE4:(r + 1) * BE4] for r in range(4)], axis=1)


def _dense_phase(xs, xd, w1t, w2t, w3t):
    grid = (E_PAD // BE,)
    return pl.pallas_call(
        _dense_body,
        grid=grid,
        in_specs=[
            pl.BlockSpec((BE // 4, 128), lambda i: (i, 0)),
            pl.BlockSpec((BE // 4, 128), lambda i: (i, 0)),
            pl.BlockSpec((20, 20), lambda i: (0, 0)),
            pl.BlockSpec((20, 20), lambda i: (0, 0)),
            pl.BlockSpec((144, 20), lambda i: (0, 0)),
        ],
        out_specs=pl.BlockSpec((BE // 4, 128), lambda i: (i, 0)),
        out_shape=jax.ShapeDtypeStruct((E_PAD // 4, 128), jnp.float32),
    )(xs, xd, w1t, w2t, w3t)


BN_BLK = 3128
BN_NBLK = ACC_ROWS // BN_BLK


def _stats_body(pa_ref, pb_ref, out_ref, acc_ref):
    i = pl.program_id(0)

    @pl.when(i == 0)
    def _():
        acc_ref[...] = jnp.zeros_like(acc_ref)

    f = pa_ref[...] + pb_ref[...]
    rows = i * BN_BLK + lax.broadcasted_iota(jnp.int32, (BN_BLK, 32), 0)
    fm = jnp.where(rows < N_NODES, f, 0.0)
    acc_ref[0:1] += jnp.sum(fm, axis=0, keepdims=True)
    acc_ref[1:2] += jnp.sum(fm * fm, axis=0, keepdims=True)

    @pl.when(i == BN_NBLK - 1)
    def _():
        out_ref[...] = acc_ref[...]


def _norm_body(pa_ref, pb_ref, st_ref, grow_ref, brow_ref, out_ref):
    f = pa_ref[...] + pb_ref[...]
    inv_n = 1.0 / float(N_NODES)
    mu = st_ref[0:1] * inv_n
    sq = st_ref[1:2] * inv_n
    var = sq - mu * mu
    lane = lax.broadcasted_iota(jnp.int32, (32, 32), 0)
    lane_t = lax.broadcasted_iota(jnp.int32, (32, 32), 1)
    vlane = (lane >= 8) & (lane < 20) & (lane_t >= 8) & (lane_t < 20)
    m3 = jnp.where(vlane & ((lane - 8) % 4 == (lane_t - 8) % 4), 1.0, 0.0)
    n2 = sq @ m3
    s_lane = lax.broadcasted_iota(jnp.int32, (1, 32), 1) < 8
    denom = jnp.sqrt(jnp.where(s_lane, var, n2) + 1e-5)
    norm = jnp.where(s_lane, f - mu, f) / denom
    res = norm * grow_ref[...] + brow_ref[...]
    sblock = (lane == lane_t) & (lane_t < 8)
    vperm = vlane & (lane - 8 == ((lane_t - 8) % 3) * 4 + (lane_t - 8) // 3)
    pmat = jnp.where(sblock | vperm, 1.0, 0.0)
    res = res @ pmat
    out_ref[...] = res[:, 0:20]


def _bn_phase(parts, grow, brow):
    pa_spec = pl.BlockSpec((BN_BLK, 32), lambda i: (i, 0))
    pb_spec = pl.BlockSpec((BN_BLK, 32), lambda i: (i + BN_NBLK, 0))
    stats = pl.pallas_call(
        _stats_body,
        grid=(BN_NBLK,),
        in_specs=[pa_spec, pb_spec],
        out_specs=pl.BlockSpec((2, 32), lambda i: (0, 0)),
        out_shape=jax.ShapeDtypeStruct((2, 32), jnp.float32),
        scratch_shapes=[pltpu.VMEM((2, 32), jnp.float32)],
    )(parts, parts)
    return pl.pallas_call(
        _norm_body,
        grid=(BN_NBLK,),
        in_specs=[
            pa_spec,
            pb_spec,
            pl.BlockSpec((2, 32), lambda i: (0, 0)),
            pl.BlockSpec((1, 32), lambda i: (0, 0)),
            pl.BlockSpec((1, 32), lambda i: (0, 0)),
        ],
        out_specs=pl.BlockSpec((BN_BLK, 20), lambda i: (i, 0)),
        out_shape=jax.ShapeDtypeStruct((N_NODES, 20), jnp.float32),
    )(parts, parts, stats, grow, brow)


def kernel(pos, batch, f_in, edge_index, W1, W2, W3, gamma_s, beta_s, gamma_v):
    src = edge_index[0]
    dst = edge_index[1]
    pad = E_PAD - N_EDGES
    src_p = jnp.concatenate([src, jnp.zeros((pad,), jnp.int32)])
    dst_p = jnp.concatenate([dst, jnp.full((pad,), DUMP_ROW, jnp.int32)])

    tab32 = jnp.concatenate(
        [pos, f_in, jnp.zeros((N_NODES, 9), jnp.float32)], axis=1)

    xs, xd = _gather_phase()(src_p, dst_p, tab32)
    xs = xs.reshape(E_PAD // 4, 128)
    xd = xd.reshape(E_PAD // 4, 128)

    w1t = (W1 * (1.0 / np.sqrt(float(NBASIS)))).T.astype(jnp.bfloat16)
    w2t = (W2 * (1.0 / np.sqrt(20.0))).T.astype(jnp.bfloat16)
    w3t = (W3 * (1.0 / np.sqrt(20.0))).T.astype(jnp.bfloat16)
    fe = _dense_phase(xs, xd, w1t, w2t, w3t).reshape(E_PAD, 32)

    zeros_acc = jnp.zeros((ACC_ROWS, 32), jnp.float32)
    parts = _scatter_phase()(dst_p, fe, zeros_acc)

    grow = jnp.concatenate(
        [gamma_s, jnp.tile(gamma_v, 3), jnp.zeros((12,), jnp.float32)]
    ).reshape(1, 32)
    brow = jnp.concatenate(
        [beta_s, jnp.zeros((24,), jnp.float32)]).reshape(1, 32)
    return _bn_phase(parts, grow, brow)


# speedup vs baseline: 12.9379x; 1.2712x over previous
import functools

import jax
import jax.numpy as jnp
import numpy as np
from jax import lax
from jax.experimental import pallas as pl
from jax.experimental.pallas import tpu as pltpu
from jax.experimental.pallas import tpu_sc as plsc

N_NODES = 50000
N_EDGES = 800000
RADIUS = 5.0
NBASIS = 20

NC, NS = 2, 16
NW = NC * NS
CB = 128
CPW = 196
E_PAD = NW * CPW * CB
NSLICE = 2
E_SL = E_PAD // NSLICE
CPW_SL = CPW // NSLICE
CPT_SL = E_SL // NC // NS // CB
ACC_ROWS = 50048
RPT = ACC_ROWS // NS
DUMP_ROW = N_NODES

BE = 2048

_SQ2 = float(np.sqrt(2.0))
_SQ3 = float(np.sqrt(3.0))
_ALPHA = float(1.0 / np.sqrt(12.0))
_EMBC = float(1.14136 * np.exp(2.0) * np.sqrt(float(NBASIS)))
_STEP = float(RADIUS / (NBASIS + 1))

@functools.cache
def _gather_phase():
    mesh = plsc.VectorSubcoreMesh(core_axis_name="c", subcore_axis_name="s")
    return functools.partial(
        pl.kernel,
        out_type=(
            jax.ShapeDtypeStruct((E_SL, 32), jnp.float32),
            jax.ShapeDtypeStruct((E_SL, 32), jnp.float32),
        ),
        mesh=mesh,
        scratch_types=[
            pltpu.VMEM((CB,), jnp.int32),
            pltpu.VMEM((CB,), jnp.int32),
            pltpu.VMEM((CB, 32), jnp.float32),
            pltpu.VMEM((CB, 32), jnp.float32),
            pltpu.SemaphoreType.DMA,
            pltpu.SemaphoreType.DMA,
        ],
        compiler_params=pltpu.CompilerParams(use_tc_tiling_on_sc=False),
    )(_gather_body)


def _gather_body(src_hbm, dst_hbm, tab32_hbm, os_hbm, od_hbm,
                 idx_s, idx_d, buf_s, buf_d, sem_a, sem_b):
    wid = lax.axis_index("s") * NC + lax.axis_index("c")

    def body(ci, _):
        base = (wid * CPW_SL + ci) * CB
        pltpu.sync_copy(src_hbm.at[pl.ds(base, CB)], idx_s)
        pltpu.sync_copy(dst_hbm.at[pl.ds(base, CB)], idx_d)
        a = pltpu.async_copy(tab32_hbm.at[idx_s], buf_s, sem_a)
        b = pltpu.async_copy(tab32_hbm.at[idx_d], buf_d, sem_b)
        a.wait()
        b.wait()
        pltpu.sync_copy(buf_s, os_hbm.at[pl.ds(base, CB)])
        pltpu.sync_copy(buf_d, od_hbm.at[pl.ds(base, CB)])
        return 0

    lax.fori_loop(0, CPW_SL, body, 0)


@functools.cache
def _scatter_phase():
    mesh = plsc.VectorSubcoreMesh(core_axis_name="c", subcore_axis_name="s")
    return functools.partial(
        pl.kernel,
        out_type=jax.ShapeDtypeStruct((NC * ACC_ROWS, 32), jnp.float32),
        mesh=mesh,
        scratch_types=[
            pltpu.VMEM((CB,), jnp.int32),
            pltpu.VMEM((CB, 32), jnp.float32),
            pltpu.VMEM_SHARED((ACC_ROWS, 32), jnp.float32),
        ],
        compiler_params=pltpu.CompilerParams(use_tc_tiling_on_sc=False),
    )(_scatter_body)


def _scatter_body(dst_hbm, fe_hbm, zeros_hbm, out_hbm, idx_v, buf, acc):
    cid = lax.axis_index("c")
    sid = lax.axis_index("s")

    pltpu.sync_copy(zeros_hbm.at[pl.ds(sid * RPT, RPT)],
                    acc.at[pl.ds(sid * RPT, RPT)])
    plsc.subcore_barrier()

    half = E_SL // NC

    def body(ci, _):
        base = cid * half + (sid * CPT_SL + ci) * CB
        pltpu.sync_copy(dst_hbm.at[pl.ds(base, CB)], idx_v)
        pltpu.sync_copy(fe_hbm.at[pl.ds(base, CB)], buf)
        pltpu.sync_copy(buf, acc.at[idx_v], add=True)
        return 0

    lax.fori_loop(0, CPT_SL, body, 0)

    plsc.subcore_barrier()
    pltpu.sync_copy(acc.at[pl.ds(sid * RPT, RPT)],
                    out_hbm.at[pl.ds(cid * ACC_ROWS + sid * RPT, RPT)])


def _dense_body(xs_ref, xd_ref, w1t_ref, w2t_ref, w3t_ref, out_ref):
    BE4 = BE // 4
    y = xs_ref[...].T
    xt = jnp.concatenate([y[32 * r:32 * r + 32] for r in range(4)], axis=1)
    z = xd_ref[...].T
    pdt = jnp.concatenate([z[32 * r:32 * r + 3] for r in range(4)], axis=1)
    vec = pdt - xt[0:3]
    r2 = vec[0:1] * vec[0:1] + vec[1:2] * vec[1:2] + vec[2:3] * vec[2:3] + 1e-12
    rinv = lax.rsqrt(r2)
    r = r2 * rinv
    y1 = _SQ3 * vec * rinv

    vals = _STEP * (1.0 + lax.broadcasted_iota(
        jnp.int32, (NBASIS, 1), 0).astype(jnp.float32))
    d = (r - vals) * (1.0 / _STEP)
    u = 1.0 - d * d
    good = u > 0.0
    emb = jnp.where(good, _EMBC * jnp.exp(-2.0 / jnp.where(good, u, 1.0)), 0.0)

    f32 = jnp.float32
    h = jnp.dot(w1t_ref[...], emb.astype(jnp.bfloat16),
                preferred_element_type=f32)
    h = (jnp.maximum(h, 0.0) * _SQ2).astype(jnp.bfloat16)
    h = jnp.dot(w2t_ref[...], h, preferred_element_type=f32)
    h = (jnp.maximum(h, 0.0) * _SQ2).astype(jnp.bfloat16)
    w = jnp.dot(w3t_ref[...], h, preferred_element_type=f32)

    s = xt[3:11]
    v = xt[11:23]

    dots = []
    for k in range(4):
        dk = (v[3 * k:3 * k + 1] * y1[0:1]
              + v[3 * k + 1:3 * k + 2] * y1[1:2]
              + v[3 * k + 2:3 * k + 3] * y1[2:3]) * (1.0 / _SQ3)
        dots.append(dk)

    out0 = s[0:1] * w[0:8]
    for i in range(1, 8):
        out0 = out0 + s[i:i + 1] * w[8 * i:8 * i + 8]
    for k in range(4):
        out0 = out0 + dots[k] * w[64 + 8 * k:72 + 8 * k]
    out0 = out0 * _ALPHA

    p = s[0:1] * w[96:100]
    for i in range(1, 8):
        p = p + s[i:i + 1] * w[96 + 4 * i:100 + 4 * i]
    q = []
    for c in range(3):
        qc = v[c:c + 1] * w[128:132]
        for k in range(1, 4):
            qc = qc + v[3 * k + c:3 * k + c + 1] * w[128 + 4 * k:132 + 4 * k]
        q.append(qc)

    rows = [out0]
    for c in range(3):
        rows.append((p * y1[c:c + 1] + q[c]) * _ALPHA)
    rows.append(jnp.zeros((12, BE), jnp.float32))
    fe = jnp.concatenate(rows, axis=0)
    ft = fe.T
    out_ref[...] = jnp.concatenate(
        [ft[r * BE4:(r + 1) * BE4] for r in range(4)], axis=1)


def _dense_phase(xs, xd, w1t, w2t, w3t):
    grid = (E_SL // BE,)
    return pl.pallas_call(
        _dense_body,
        grid=grid,
        in_specs=[
            pl.BlockSpec((BE // 4, 128), lambda i: (i, 0)),
            pl.BlockSpec((BE // 4, 128), lambda i: (i, 0)),
            pl.BlockSpec((20, 20), lambda i: (0, 0)),
            pl.BlockSpec((20, 20), lambda i: (0, 0)),
            pl.BlockSpec((144, 20), lambda i: (0, 0)),
        ],
        out_specs=pl.BlockSpec((BE // 4, 128), lambda i: (i, 0)),
        out_shape=jax.ShapeDtypeStruct((E_SL // 4, 128), jnp.float32),
    )(xs, xd, w1t, w2t, w3t)


BN_BLK = 3128
BN_NBLK = ACC_ROWS // BN_BLK


def _stats_body(pa0_ref, pb0_ref, pa1_ref, pb1_ref, out_ref, acc_ref):
    i = pl.program_id(0)

    @pl.when(i == 0)
    def _():
        acc_ref[...] = jnp.zeros_like(acc_ref)

    f = (pa0_ref[...] + pb0_ref[...]) + (pa1_ref[...] + pb1_ref[...])
    rows = i * BN_BLK + lax.broadcasted_iota(jnp.int32, (BN_BLK, 32), 0)
    fm = jnp.where(rows < N_NODES, f, 0.0)
    acc_ref[0:1] += jnp.sum(fm, axis=0, keepdims=True)
    acc_ref[1:2] += jnp.sum(fm * fm, axis=0, keepdims=True)

    @pl.when(i == BN_NBLK - 1)
    def _():
        out_ref[...] = acc_ref[...]


def _norm_body(pa0_ref, pb0_ref, pa1_ref, pb1_ref, st_ref, grow_ref, brow_ref,
               out_ref):
    f = (pa0_ref[...] + pb0_ref[...]) + (pa1_ref[...] + pb1_ref[...])
    inv_n = 1.0 / float(N_NODES)
    mu = st_ref[0:1] * inv_n
    sq = st_ref[1:2] * inv_n
    var = sq - mu * mu
    lane = lax.broadcasted_iota(jnp.int32, (32, 32), 0)
    lane_t = lax.broadcasted_iota(jnp.int32, (32, 32), 1)
    vlane = (lane >= 8) & (lane < 20) & (lane_t >= 8) & (lane_t < 20)
    m3 = jnp.where(vlane & ((lane - 8) % 4 == (lane_t - 8) % 4), 1.0, 0.0)
    n2 = sq @ m3
    s_lane = lax.broadcasted_iota(jnp.int32, (1, 32), 1) < 8
    denom = jnp.sqrt(jnp.where(s_lane, var, n2) + 1e-5)
    norm = jnp.where(s_lane, f - mu, f) / denom
    res = norm * grow_ref[...] + brow_ref[...]
    sblock = (lane == lane_t) & (lane_t < 8)
    vperm = vlane & (lane - 8 == ((lane_t - 8) % 3) * 4 + (lane_t - 8) // 3)
    pmat = jnp.where(sblock | vperm, 1.0, 0.0)
    res = res @ pmat
    out_ref[...] = res[:, 0:20]


def _bn_phase(parts0, parts1, grow, brow):
    pa_spec = pl.BlockSpec((BN_BLK, 32), lambda i: (i, 0))
    pb_spec = pl.BlockSpec((BN_BLK, 32), lambda i: (i + BN_NBLK, 0))
    stats = pl.pallas_call(
        _stats_body,
        grid=(BN_NBLK,),
        in_specs=[pa_spec, pb_spec, pa_spec, pb_spec],
        out_specs=pl.BlockSpec((2, 32), lambda i: (0, 0)),
        out_shape=jax.ShapeDtypeStruct((2, 32), jnp.float32),
        scratch_shapes=[pltpu.VMEM((2, 32), jnp.float32)],
    )(parts0, parts0, parts1, parts1)
    return pl.pallas_call(
        _norm_body,
        grid=(BN_NBLK,),
        in_specs=[
            pa_spec,
            pb_spec,
            pa_spec,
            pb_spec,
            pl.BlockSpec((2, 32), lambda i: (0, 0)),
            pl.BlockSpec((1, 32), lambda i: (0, 0)),
            pl.BlockSpec((1, 32), lambda i: (0, 0)),
        ],
        out_specs=pl.BlockSpec((BN_BLK, 20), lambda i: (i, 0)),
        out_shape=jax.ShapeDtypeStruct((N_NODES, 20), jnp.float32),
    )(parts0, parts0, parts1, parts1, stats, grow, brow)


def kernel(pos, batch, f_in, edge_index, W1, W2, W3, gamma_s, beta_s, gamma_v):
    src = edge_index[0]
    dst = edge_index[1]
    pad = E_PAD - N_EDGES
    src_p = jnp.concatenate([src, jnp.zeros((pad,), jnp.int32)])
    dst_p = jnp.concatenate([dst, jnp.full((pad,), DUMP_ROW, jnp.int32)])

    tab32 = jnp.concatenate(
        [pos, f_in, jnp.zeros((N_NODES, 9), jnp.float32)], axis=1)

    w1t = (W1 * (1.0 / np.sqrt(float(NBASIS)))).T.astype(jnp.bfloat16)
    w2t = (W2 * (1.0 / np.sqrt(20.0))).T.astype(jnp.bfloat16)
    w3t = (W3 * (1.0 / np.sqrt(20.0))).T.astype(jnp.bfloat16)
    zeros_acc = jnp.zeros((ACC_ROWS, 32), jnp.float32)

    parts = []
    srcs = [lax.slice(src_p, (k * E_SL,), ((k + 1) * E_SL,))
            for k in range(NSLICE)]
    dsts = [lax.slice(dst_p, (k * E_SL,), ((k + 1) * E_SL,))
            for k in range(NSLICE)]
    gs = [_gather_phase()(srcs[k], dsts[k], tab32) for k in range(NSLICE)]
    for k in range(NSLICE):
        xs, xd = gs[k]
        xs = xs.reshape(E_SL // 4, 128)
        xd = xd.reshape(E_SL // 4, 128)
        fe = _dense_phase(xs, xd, w1t, w2t, w3t).reshape(E_SL, 32)
        parts.append(_scatter_phase()(dsts[k], fe, zeros_acc))

    grow = jnp.concatenate(
        [gamma_s, jnp.tile(gamma_v, 3), jnp.zeros((12,), jnp.float32)]
    ).reshape(1, 32)
    brow = jnp.concatenate(
        [beta_s, jnp.zeros((24,), jnp.float32)]).reshape(1, 32)
    return _bn_phase(parts[0], parts[1], grow, brow)


# speedup vs baseline: 12.9634x; 1.0020x over previous
import functools

import jax
import jax.numpy as jnp
import numpy as np
from jax import lax
from jax.experimental import pallas as pl
from jax.experimental.pallas import tpu as pltpu
from jax.experimental.pallas import tpu_sc as plsc

N_NODES = 50000
N_EDGES = 800000
RADIUS = 5.0
NBASIS = 20

NC, NS = 2, 16
NW = NC * NS
CB = 128
CPW = 196
E_PAD = NW * CPW * CB
NSLICE = 4
E_SL = E_PAD // NSLICE
CPW_SL = CPW // NSLICE
CPT_SL = E_SL // NC // NS // CB
ACC_ROWS = 50048
RPT = ACC_ROWS // NS
DUMP_ROW = N_NODES

BE = 2048

_SQ2 = float(np.sqrt(2.0))
_SQ3 = float(np.sqrt(3.0))
_ALPHA = float(1.0 / np.sqrt(12.0))
_EMBC = float(1.14136 * np.exp(2.0) * np.sqrt(float(NBASIS)))
_STEP = float(RADIUS / (NBASIS + 1))

@functools.cache
def _gather_phase():
    mesh = plsc.VectorSubcoreMesh(core_axis_name="c", subcore_axis_name="s")
    return functools.partial(
        pl.kernel,
        out_type=(
            jax.ShapeDtypeStruct((E_SL, 32), jnp.float32),
            jax.ShapeDtypeStruct((E_SL, 32), jnp.float32),
        ),
        mesh=mesh,
        scratch_types=[
            pltpu.VMEM((CB,), jnp.int32),
            pltpu.VMEM((CB,), jnp.int32),
            pltpu.VMEM((CB, 32), jnp.float32),
            pltpu.VMEM((CB, 32), jnp.float32),
            pltpu.SemaphoreType.DMA,
            pltpu.SemaphoreType.DMA,
        ],
        compiler_params=pltpu.CompilerParams(use_tc_tiling_on_sc=False),
    )(_gather_body)


def _gather_body(src_hbm, dst_hbm, tab32_hbm, os_hbm, od_hbm,
                 idx_s, idx_d, buf_s, buf_d, sem_a, sem_b):
    wid = lax.axis_index("s") * NC + lax.axis_index("c")

    def body(ci, _):
        base = (wid * CPW_SL + ci) * CB
        pltpu.sync_copy(src_hbm.at[pl.ds(base, CB)], idx_s)
        pltpu.sync_copy(dst_hbm.at[pl.ds(base, CB)], idx_d)
        a = pltpu.async_copy(tab32_hbm.at[idx_s], buf_s, sem_a)
        b = pltpu.async_copy(tab32_hbm.at[idx_d], buf_d, sem_b)
        a.wait()
        b.wait()
        pltpu.sync_copy(buf_s, os_hbm.at[pl.ds(base, CB)])
        pltpu.sync_copy(buf_d, od_hbm.at[pl.ds(base, CB)])
        return 0

    lax.fori_loop(0, CPW_SL, body, 0)


@functools.cache
def _scatter_phase():
    mesh = plsc.VectorSubcoreMesh(core_axis_name="c", subcore_axis_name="s")
    return functools.partial(
        pl.kernel,
        out_type=jax.ShapeDtypeStruct((NC * ACC_ROWS, 32), jnp.float32),
        mesh=mesh,
        scratch_types=[
            pltpu.VMEM((CB,), jnp.int32),
            pltpu.VMEM((CB, 32), jnp.float32),
            pltpu.VMEM_SHARED((ACC_ROWS, 32), jnp.float32),
        ],
        compiler_params=pltpu.CompilerParams(use_tc_tiling_on_sc=False),
    )(_scatter_body)


def _scatter_body(dst_hbm, fe_hbm, zeros_hbm, out_hbm, idx_v, buf, acc):
    cid = lax.axis_index("c")
    sid = lax.axis_index("s")

    pltpu.sync_copy(zeros_hbm.at[pl.ds(sid * RPT, RPT)],
                    acc.at[pl.ds(sid * RPT, RPT)])
    plsc.subcore_barrier()

    half = E_SL // NC

    def body(ci, _):
        base = cid * half + (sid * CPT_SL + ci) * CB
        pltpu.sync_copy(dst_hbm.at[pl.ds(base, CB)], idx_v)
        pltpu.sync_copy(fe_hbm.at[pl.ds(base, CB)], buf)
        pltpu.sync_copy(buf, acc.at[idx_v], add=True)
        return 0

    lax.fori_loop(0, CPT_SL, body, 0)

    plsc.subcore_barrier()
    pltpu.sync_copy(acc.at[pl.ds(sid * RPT, RPT)],
                    out_hbm.at[pl.ds(cid * ACC_ROWS + sid * RPT, RPT)])


def _dense_body(xs_ref, xd_ref, w1t_ref, w2t_ref, w3t_ref, out_ref):
    BE4 = BE // 4
    y = xs_ref[...].T
    xt = jnp.concatenate([y[32 * r:32 * r + 32] for r in range(4)], axis=1)
    z = xd_ref[...].T
    pdt = jnp.concatenate([z[32 * r:32 * r + 3] for r in range(4)], axis=1)
    vec = pdt - xt[0:3]
    r2 = vec[0:1] * vec[0:1] + vec[1:2] * vec[1:2] + vec[2:3] * vec[2:3] + 1e-12
    rinv = lax.rsqrt(r2)
    r = r2 * rinv
    y1 = _SQ3 * vec * rinv

    vals = _STEP * (1.0 + lax.broadcasted_iota(
        jnp.int32, (NBASIS, 1), 0).astype(jnp.float32))
    d = (r - vals) * (1.0 / _STEP)
    u = 1.0 - d * d
    good = u > 0.0
    emb = jnp.where(good, _EMBC * jnp.exp(-2.0 / jnp.where(good, u, 1.0)), 0.0)

    f32 = jnp.float32
    h = jnp.dot(w1t_ref[...], emb.astype(jnp.bfloat16),
                preferred_element_type=f32)
    h = (jnp.maximum(h, 0.0) * _SQ2).astype(jnp.bfloat16)
    h = jnp.dot(w2t_ref[...], h, preferred_element_type=f32)
    h = (jnp.maximum(h, 0.0) * _SQ2).astype(jnp.bfloat16)
    w = jnp.dot(w3t_ref[...], h, preferred_element_type=f32)

    s = xt[3:11]
    v = xt[11:23]

    dots = []
    for k in range(4):
        dk = (v[3 * k:3 * k + 1] * y1[0:1]
              + v[3 * k + 1:3 * k + 2] * y1[1:2]
              + v[3 * k + 2:3 * k + 3] * y1[2:3]) * (1.0 / _SQ3)
        dots.append(dk)

    out0 = s[0:1] * w[0:8]
    for i in range(1, 8):
        out0 = out0 + s[i:i + 1] * w[8 * i:8 * i + 8]
    for k in range(4):
        out0 = out0 + dots[k] * w[64 + 8 * k:72 + 8 * k]
    out0 = out0 * _ALPHA

    p = s[0:1] * w[96:100]
    for i in range(1, 8):
        p = p + s[i:i + 1] * w[96 + 4 * i:100 + 4 * i]
    q = []
    for c in range(3):
        qc = v[c:c + 1] * w[128:132]
        for k in range(1, 4):
            qc = qc + v[3 * k + c:3 * k + c + 1] * w[128 + 4 * k:132 + 4 * k]
        q.append(qc)

    rows = [out0]
    for c in range(3):
        rows.append((p * y1[c:c + 1] + q[c]) * _ALPHA)
    rows.append(jnp.zeros((12, BE), jnp.float32))
    fe = jnp.concatenate(rows, axis=0)
    ft = fe.T
    out_ref[...] = jnp.concatenate(
        [ft[r * BE4:(r + 1) * BE4] for r in range(4)], axis=1)


def _dense_phase(xs, xd, w1t, w2t, w3t):
    grid = (E_SL // BE,)
    return pl.pallas_call(
        _dense_body,
        grid=grid,
        in_specs=[
            pl.BlockSpec((BE // 4, 128), lambda i: (i, 0)),
            pl.BlockSpec((BE // 4, 128), lambda i: (i, 0)),
            pl.BlockSpec((20, 20), lambda i: (0, 0)),
            pl.BlockSpec((20, 20), lambda i: (0, 0)),
            pl.BlockSpec((144, 20), lambda i: (0, 0)),
        ],
        out_specs=pl.BlockSpec((BE // 4, 128), lambda i: (i, 0)),
        out_shape=jax.ShapeDtypeStruct((E_SL // 4, 128), jnp.float32),
    )(xs, xd, w1t, w2t, w3t)


BN_BLK = 3128
BN_NBLK = ACC_ROWS // BN_BLK


def _stats_body(*args):
    part_refs, (out_ref, acc_ref) = args[:-2], args[-2:]
    i = pl.program_id(0)

    @pl.when(i == 0)
    def _():
        acc_ref[...] = jnp.zeros_like(acc_ref)

    f = sum(ref[...] for ref in part_refs)
    rows = i * BN_BLK + lax.broadcasted_iota(jnp.int32, (BN_BLK, 32), 0)
    fm = jnp.where(rows < N_NODES, f, 0.0)
    acc_ref[0:1] += jnp.sum(fm, axis=0, keepdims=True)
    acc_ref[1:2] += jnp.sum(fm * fm, axis=0, keepdims=True)

    @pl.when(i == BN_NBLK - 1)
    def _():
        out_ref[...] = acc_ref[...]


def _norm_body(*args):
    part_refs = args[:-4]
    st_ref, grow_ref, brow_ref, out_ref = args[-4:]
    f = sum(ref[...] for ref in part_refs)
    inv_n = 1.0 / float(N_NODES)
    mu = st_ref[0:1] * inv_n
    sq = st_ref[1:2] * inv_n
    var = sq - mu * mu
    lane = lax.broadcasted_iota(jnp.int32, (32, 32), 0)
    lane_t = lax.broadcasted_iota(jnp.int32, (32, 32), 1)
    vlane = (lane >= 8) & (lane < 20) & (lane_t >= 8) & (lane_t < 20)
    m3 = jnp.where(vlane & ((lane - 8) % 4 == (lane_t - 8) % 4), 1.0, 0.0)
    n2 = sq @ m3
    s_lane = lax.broadcasted_iota(jnp.int32, (1, 32), 1) < 8
    denom = jnp.sqrt(jnp.where(s_lane, var, n2) + 1e-5)
    norm = jnp.where(s_lane, f - mu, f) / denom
    res = norm * grow_ref[...] + brow_ref[...]
    sblock = (lane == lane_t) & (lane_t < 8)
    vperm = vlane & (lane - 8 == ((lane_t - 8) % 3) * 4 + (lane_t - 8) // 3)
    pmat = jnp.where(sblock | vperm, 1.0, 0.0)
    res = res @ pmat
    out_ref[...] = res[:, 0:20]


def _bn_phase(parts, grow, brow):
    pa_spec = pl.BlockSpec((BN_BLK, 32), lambda i: (i, 0))
    pb_spec = pl.BlockSpec((BN_BLK, 32), lambda i: (i + BN_NBLK, 0))
    part_specs = [s for _ in parts for s in (pa_spec, pb_spec)]
    part_args = [x for pt in parts for x in (pt, pt)]
    stats = pl.pallas_call(
        _stats_body,
        grid=(BN_NBLK,),
        in_specs=part_specs,
        out_specs=pl.BlockSpec((2, 32), lambda i: (0, 0)),
        out_shape=jax.ShapeDtypeStruct((2, 32), jnp.float32),
        scratch_shapes=[pltpu.VMEM((2, 32), jnp.float32)],
    )(*part_args)
    return pl.pallas_call(
        _norm_body,
        grid=(BN_NBLK,),
        in_specs=part_specs + [
            pl.BlockSpec((2, 32), lambda i: (0, 0)),
            pl.BlockSpec((1, 32), lambda i: (0, 0)),
            pl.BlockSpec((1, 32), lambda i: (0, 0)),
        ],
        out_specs=pl.BlockSpec((BN_BLK, 20), lambda i: (i, 0)),
        out_shape=jax.ShapeDtypeStruct((N_NODES, 20), jnp.float32),
    )(*part_args, stats, grow, brow)


def kernel(pos, batch, f_in, edge_index, W1, W2, W3, gamma_s, beta_s, gamma_v):
    src = edge_index[0]
    dst = edge_index[1]
    pad = E_PAD - N_EDGES
    src_p = jnp.concatenate([src, jnp.zeros((pad,), jnp.int32)])
    dst_p = jnp.concatenate([dst, jnp.full((pad,), DUMP_ROW, jnp.int32)])

    tab32 = jnp.concatenate(
        [pos, f_in, jnp.zeros((N_NODES, 9), jnp.float32)], axis=1)

    w1t = (W1 * (1.0 / np.sqrt(float(NBASIS)))).T.astype(jnp.bfloat16)
    w2t = (W2 * (1.0 / np.sqrt(20.0))).T.astype(jnp.bfloat16)
    w3t = (W3 * (1.0 / np.sqrt(20.0))).T.astype(jnp.bfloat16)
    zeros_acc = jnp.zeros((ACC_ROWS, 32), jnp.float32)

    parts = []
    srcs = [lax.slice(src_p, (k * E_SL,), ((k + 1) * E_SL,))
            for k in range(NSLICE)]
    dsts = [lax.slice(dst_p, (k * E_SL,), ((k + 1) * E_SL,))
            for k in range(NSLICE)]
    gs = [_gather_phase()(srcs[k], dsts[k], tab32) for k in range(NSLICE)]
    for k in range(NSLICE):
        xs, xd = gs[k]
        xs = xs.reshape(E_SL // 4, 128)
        xd = xd.reshape(E_SL // 4, 128)
        fe = _dense_phase(xs, xd, w1t, w2t, w3t).reshape(E_SL, 32)
        parts.append(_scatter_phase()(dsts[k], fe, zeros_acc))

    grow = jnp.concatenate(
        [gamma_s, jnp.tile(gamma_v, 3), jnp.zeros((12,), jnp.float32)]
    ).reshape(1, 32)
    brow = jnp.concatenate(
        [beta_s, jnp.zeros((24,), jnp.float32)]).reshape(1, 32)
    return _bn_phase(parts, grow, brow)


# speedup vs baseline: 13.8208x; 1.0661x over previous
import functools

import jax
import jax.numpy as jnp
import numpy as np
from jax import lax
from jax.experimental import pallas as pl
from jax.experimental.pallas import tpu as pltpu
from jax.experimental.pallas import tpu_sc as plsc

N_NODES = 50000
N_EDGES = 800000
RADIUS = 5.0
NBASIS = 20

NC, NS = 2, 16
NW = NC * NS
CB = 128
CPW = 196
E_PAD = NW * CPW * CB
NSLICE = 4
E_SL = E_PAD // NSLICE
CPW_SL = CPW // NSLICE
CPT_SL = E_SL // NC // NS // CB
ACC_ROWS = 50048
RPT = ACC_ROWS // NS
DUMP_ROW = N_NODES

BE = 2048

_SQ2 = float(np.sqrt(2.0))
_SQ3 = float(np.sqrt(3.0))
_ALPHA = float(1.0 / np.sqrt(12.0))
_EMBC = float(1.14136 * np.exp(2.0) * np.sqrt(float(NBASIS)))
_STEP = float(RADIUS / (NBASIS + 1))

@functools.cache
def _gather_phase():
    mesh = plsc.VectorSubcoreMesh(core_axis_name="c", subcore_axis_name="s")
    return functools.partial(
        pl.kernel,
        out_type=(
            jax.ShapeDtypeStruct((E_SL, 32), jnp.float32),
            jax.ShapeDtypeStruct((E_SL, 32), jnp.float32),
        ),
        mesh=mesh,
        scratch_types=[
            pltpu.VMEM((CB,), jnp.int32),
            pltpu.VMEM((CB,), jnp.int32),
            pltpu.VMEM((CB, 32), jnp.float32),
            pltpu.VMEM((CB, 32), jnp.float32),
            pltpu.SemaphoreType.DMA,
            pltpu.SemaphoreType.DMA,
        ],
        compiler_params=pltpu.CompilerParams(use_tc_tiling_on_sc=False),
    )(_gather_body)


def _gather_body(src_hbm, dst_hbm, tab32_hbm, os_hbm, od_hbm,
                 idx_s, idx_d, buf_s, buf_d, sem_a, sem_b):
    wid = lax.axis_index("s") * NC + lax.axis_index("c")

    def body(ci, _):
        base = (wid * CPW_SL + ci) * CB
        pltpu.sync_copy(src_hbm.at[pl.ds(base, CB)], idx_s)
        pltpu.sync_copy(dst_hbm.at[pl.ds(base, CB)], idx_d)
        a = pltpu.async_copy(tab32_hbm.at[idx_s], buf_s, sem_a)
        b = pltpu.async_copy(tab32_hbm.at[idx_d], buf_d, sem_b)
        a.wait()
        b.wait()
        pltpu.sync_copy(buf_s, os_hbm.at[pl.ds(base, CB)])
        pltpu.sync_copy(buf_d, od_hbm.at[pl.ds(base, CB)])
        return 0

    lax.fori_loop(0, CPW_SL, body, 0)


@functools.cache
def _scatter_phase():
    mesh = plsc.VectorSubcoreMesh(core_axis_name="c", subcore_axis_name="s")
    return functools.partial(
        pl.kernel,
        out_type=jax.ShapeDtypeStruct((NC * ACC_ROWS, 32), jnp.float32),
        mesh=mesh,
        scratch_types=[
            pltpu.VMEM((CB,), jnp.int32),
            pltpu.VMEM((CB, 32), jnp.float32),
            pltpu.VMEM_SHARED((ACC_ROWS, 32), jnp.float32),
        ],
        compiler_params=pltpu.CompilerParams(use_tc_tiling_on_sc=False),
    )(_scatter_body)


def _scatter_body(dst_hbm, fe_hbm, init_hbm, out_hbm, idx_v, buf, acc):
    cid = lax.axis_index("c")
    sid = lax.axis_index("s")

    pltpu.sync_copy(init_hbm.at[pl.ds(cid * ACC_ROWS + sid * RPT, RPT)],
                    acc.at[pl.ds(sid * RPT, RPT)])
    plsc.subcore_barrier()

    half = E_SL // NC

    def body(ci, _):
        base = cid * half + (sid * CPT_SL + ci) * CB
        pltpu.sync_copy(dst_hbm.at[pl.ds(base, CB)], idx_v)
        pltpu.sync_copy(fe_hbm.at[pl.ds(base, CB)], buf)
        pltpu.sync_copy(buf, acc.at[idx_v], add=True)
        return 0

    lax.fori_loop(0, CPT_SL, body, 0)

    plsc.subcore_barrier()
    pltpu.sync_copy(acc.at[pl.ds(sid * RPT, RPT)],
                    out_hbm.at[pl.ds(cid * ACC_ROWS + sid * RPT, RPT)])


def _dense_body(xs_ref, xd_ref, w1t_ref, w2t_ref, w3t_ref, out_ref):
    BE4 = BE // 4
    y = xs_ref[...].T
    xt = jnp.concatenate([y[32 * r:32 * r + 32] for r in range(4)], axis=1)
    z = xd_ref[...].T
    pdt = jnp.concatenate([z[32 * r:32 * r + 3] for r in range(4)], axis=1)
    vec = pdt - xt[0:3]
    r2 = vec[0:1] * vec[0:1] + vec[1:2] * vec[1:2] + vec[2:3] * vec[2:3] + 1e-12
    rinv = lax.rsqrt(r2)
    r = r2 * rinv
    y1 = _SQ3 * vec * rinv

    vals = _STEP * (1.0 + lax.broadcasted_iota(
        jnp.int32, (NBASIS, 1), 0).astype(jnp.float32))
    d = (r - vals) * (1.0 / _STEP)
    u = 1.0 - d * d
    good = u > 0.0
    emb = jnp.where(good, _EMBC * jnp.exp(-2.0 / jnp.where(good, u, 1.0)), 0.0)

    f32 = jnp.float32
    h = jnp.dot(w1t_ref[...], emb.astype(jnp.bfloat16),
                preferred_element_type=f32)
    h = (jnp.maximum(h, 0.0) * _SQ2).astype(jnp.bfloat16)
    h = jnp.dot(w2t_ref[...], h, preferred_element_type=f32)
    h = (jnp.maximum(h, 0.0) * _SQ2).astype(jnp.bfloat16)
    w = jnp.dot(w3t_ref[...], h, preferred_element_type=f32)

    s = xt[3:11]
    v = xt[11:23]

    dots = []
    for k in range(4):
        dk = (v[3 * k:3 * k + 1] * y1[0:1]
              + v[3 * k + 1:3 * k + 2] * y1[1:2]
              + v[3 * k + 2:3 * k + 3] * y1[2:3]) * (1.0 / _SQ3)
        dots.append(dk)

    out0 = s[0:1] * w[0:8]
    for i in range(1, 8):
        out0 = out0 + s[i:i + 1] * w[8 * i:8 * i + 8]
    for k in range(4):
        out0 = out0 + dots[k] * w[64 + 8 * k:72 + 8 * k]
    out0 = out0 * _ALPHA

    p = s[0:1] * w[96:100]
    for i in range(1, 8):
        p = p + s[i:i + 1] * w[96 + 4 * i:100 + 4 * i]
    q = []
    for c in range(3):
        qc = v[c:c + 1] * w[128:132]
        for k in range(1, 4):
            qc = qc + v[3 * k + c:3 * k + c + 1] * w[128 + 4 * k:132 + 4 * k]
        q.append(qc)

    rows = [out0]
    for c in range(3):
        rows.append((p * y1[c:c + 1] + q[c]) * _ALPHA)
    rows.append(jnp.zeros((12, BE), jnp.float32))
    fe = jnp.concatenate(rows, axis=0)
    ft = fe.T
    out_ref[...] = jnp.concatenate(
        [ft[r * BE4:(r + 1) * BE4] for r in range(4)], axis=1)


def _dense_phase(xs, xd, w1t, w2t, w3t):
    grid = (E_SL // BE,)
    return pl.pallas_call(
        _dense_body,
        grid=grid,
        in_specs=[
            pl.BlockSpec((BE // 4, 128), lambda i: (i, 0)),
            pl.BlockSpec((BE // 4, 128), lambda i: (i, 0)),
            pl.BlockSpec((20, 20), lambda i: (0, 0)),
            pl.BlockSpec((20, 20), lambda i: (0, 0)),
            pl.BlockSpec((144, 20), lambda i: (0, 0)),
        ],
        out_specs=pl.BlockSpec((BE // 4, 128), lambda i: (i, 0)),
        out_shape=jax.ShapeDtypeStruct((E_SL // 4, 128), jnp.float32),
    )(xs, xd, w1t, w2t, w3t)


BN_BLK = 3128
BN_NBLK = ACC_ROWS // BN_BLK


def _stats_body(*args):
    part_refs, (out_ref, acc_ref) = args[:-2], args[-2:]
    i = pl.program_id(0)

    @pl.when(i == 0)
    def _():
        acc_ref[...] = jnp.zeros_like(acc_ref)

    f = sum(ref[...] for ref in part_refs)
    rows = i * BN_BLK + lax.broadcasted_iota(jnp.int32, (BN_BLK, 32), 0)
    fm = jnp.where(rows < N_NODES, f, 0.0)
    acc_ref[0:1] += jnp.sum(fm, axis=0, keepdims=True)
    acc_ref[1:2] += jnp.sum(fm * fm, axis=0, keepdims=True)

    @pl.when(i == BN_NBLK - 1)
    def _():
        out_ref[...] = acc_ref[...]


def _norm_body(*args):
    part_refs = args[:-4]
    st_ref, grow_ref, brow_ref, out_ref = args[-4:]
    f = sum(ref[...] for ref in part_refs)
    inv_n = 1.0 / float(N_NODES)
    mu = st_ref[0:1] * inv_n
    sq = st_ref[1:2] * inv_n
    var = sq - mu * mu
    lane = lax.broadcasted_iota(jnp.int32, (32, 32), 0)
    lane_t = lax.broadcasted_iota(jnp.int32, (32, 32), 1)
    vlane = (lane >= 8) & (lane < 20) & (lane_t >= 8) & (lane_t < 20)
    m3 = jnp.where(vlane & ((lane - 8) % 4 == (lane_t - 8) % 4), 1.0, 0.0)
    n2 = sq @ m3
    s_lane = lax.broadcasted_iota(jnp.int32, (1, 32), 1) < 8
    denom = jnp.sqrt(jnp.where(s_lane, var, n2) + 1e-5)
    norm = jnp.where(s_lane, f - mu, f) / denom
    res = norm * grow_ref[...] + brow_ref[...]
    sblock = (lane == lane_t) & (lane_t < 8)
    vperm = vlane & (lane - 8 == ((lane_t - 8) % 3) * 4 + (lane_t - 8) // 3)
    pmat = jnp.where(sblock | vperm, 1.0, 0.0)
    res = res @ pmat
    out_ref[...] = res[:, 0:20]


def _bn_phase(parts, grow, brow):
    pa_spec = pl.BlockSpec((BN_BLK, 32), lambda i: (i, 0))
    pb_spec = pl.BlockSpec((BN_BLK, 32), lambda i: (i + BN_NBLK, 0))
    part_specs = [s for _ in parts for s in (pa_spec, pb_spec)]
    part_args = [x for pt in parts for x in (pt, pt)]
    stats = pl.pallas_call(
        _stats_body,
        grid=(BN_NBLK,),
        in_specs=part_specs,
        out_specs=pl.BlockSpec((2, 32), lambda i: (0, 0)),
        out_shape=jax.ShapeDtypeStruct((2, 32), jnp.float32),
        scratch_shapes=[pltpu.VMEM((2, 32), jnp.float32)],
    )(*part_args)
    return pl.pallas_call(
        _norm_body,
        grid=(BN_NBLK,),
        in_specs=part_specs + [
            pl.BlockSpec((2, 32), lambda i: (0, 0)),
            pl.BlockSpec((1, 32), lambda i: (0, 0)),
            pl.BlockSpec((1, 32), lambda i: (0, 0)),
        ],
        out_specs=pl.BlockSpec((BN_BLK, 20), lambda i: (i, 0)),
        out_shape=jax.ShapeDtypeStruct((N_NODES, 20), jnp.float32),
    )(*part_args, stats, grow, brow)


def kernel(pos, batch, f_in, edge_index, W1, W2, W3, gamma_s, beta_s, gamma_v):
    src = edge_index[0]
    dst = edge_index[1]
    pad = E_PAD - N_EDGES
    src_p = jnp.concatenate([src, jnp.zeros((pad,), jnp.int32)])
    dst_p = jnp.concatenate([dst, jnp.full((pad,), DUMP_ROW, jnp.int32)])

    tab32 = jnp.concatenate(
        [pos, f_in, jnp.zeros((N_NODES, 9), jnp.float32)], axis=1)

    w1t = (W1 * (1.0 / np.sqrt(float(NBASIS)))).T.astype(jnp.bfloat16)
    w2t = (W2 * (1.0 / np.sqrt(20.0))).T.astype(jnp.bfloat16)
    w3t = (W3 * (1.0 / np.sqrt(20.0))).T.astype(jnp.bfloat16)
    zeros_acc = jnp.zeros((NC * ACC_ROWS, 32), jnp.float32)

    srcs = [lax.slice(src_p, (k * E_SL,), ((k + 1) * E_SL,))
            for k in range(NSLICE)]
    dsts = [lax.slice(dst_p, (k * E_SL,), ((k + 1) * E_SL,))
            for k in range(NSLICE)]
    gs = [_gather_phase()(srcs[k], dsts[k], tab32) for k in range(NSLICE)]
    running = zeros_acc
    for k in range(NSLICE):
        xs, xd = gs[k]
        xs = xs.reshape(E_SL // 4, 128)
        xd = xd.reshape(E_SL // 4, 128)
        fe = _dense_phase(xs, xd, w1t, w2t, w3t).reshape(E_SL, 32)
        running = _scatter_phase()(dsts[k], fe, running)
    parts = [running]

    grow = jnp.concatenate(
        [gamma_s, jnp.tile(gamma_v, 3), jnp.zeros((12,), jnp.float32)]
    ).reshape(1, 32)
    brow = jnp.concatenate(
        [beta_s, jnp.zeros((24,), jnp.float32)]).reshape(1, 32)
    return _bn_phase(parts, grow, brow)


# speedup vs baseline: 18.1747x; 1.3150x over previous
import functools

import jax
import jax.numpy as jnp
import numpy as np
from jax import lax
from jax.experimental import pallas as pl
from jax.experimental.pallas import tpu as pltpu
from jax.experimental.pallas import tpu_sc as plsc

N_NODES = 50000
N_EDGES = 800000
RADIUS = 5.0
NBASIS = 20

NC, NS = 2, 16
NW = NC * NS
CB = 128
CPW = 196
E_PAD = NW * CPW * CB
NSLICE = 4
GRP = 7
E_SL = E_PAD // NSLICE
CPW_SL = CPW // NSLICE
CPT_SL = E_SL // NC // NS // CB
ACC_ROWS = 50048
RPT = ACC_ROWS // NS
DUMP_ROW = N_NODES

BE = 2048

_SQ2 = float(np.sqrt(2.0))
_SQ3 = float(np.sqrt(3.0))
_ALPHA = float(1.0 / np.sqrt(12.0))
_EMBC = float(1.14136 * np.exp(2.0) * np.sqrt(float(NBASIS)))
_STEP = float(RADIUS / (NBASIS + 1))

@functools.cache
def _gather_phase():
    mesh = plsc.VectorSubcoreMesh(core_axis_name="c", subcore_axis_name="s")
    return functools.partial(
        pl.kernel,
        out_type=(
            jax.ShapeDtypeStruct((E_SL, 32), jnp.float32),
            jax.ShapeDtypeStruct((E_SL, 32), jnp.float32),
        ),
        mesh=mesh,
        scratch_types=[
            pltpu.VMEM((GRP, CB), jnp.int32),
            pltpu.VMEM((GRP, CB), jnp.int32),
            pltpu.VMEM((GRP, CB, 32), jnp.float32),
            pltpu.VMEM((GRP, CB, 32), jnp.float32),
            pltpu.SemaphoreType.DMA,
            pltpu.SemaphoreType.DMA,
            pltpu.SemaphoreType.DMA,
        ],
        compiler_params=pltpu.CompilerParams(use_tc_tiling_on_sc=False),
    )(_gather_body)


def _gather_body(src_hbm, dst_hbm, tab32_hbm, os_hbm, od_hbm,
                 idx_s, idx_d, buf_s, buf_d, sem_i, sem_g, sem_w):
    wid = lax.axis_index("s") * NC + lax.axis_index("c")

    def body(g, _):
        base0 = (wid * CPW_SL + g * GRP) * CB
        pend = []
        for j in range(GRP):
            pend.append(pltpu.async_copy(
                src_hbm.at[pl.ds(base0 + j * CB, CB)], idx_s.at[j], sem_i))
            pend.append(pltpu.async_copy(
                dst_hbm.at[pl.ds(base0 + j * CB, CB)], idx_d.at[j], sem_i))
        for dsc in pend:
            dsc.wait()
        pend = []
        for j in range(GRP):
            pend.append(pltpu.async_copy(
                tab32_hbm.at[idx_s.at[j]], buf_s.at[j], sem_g))
            pend.append(pltpu.async_copy(
                tab32_hbm.at[idx_d.at[j]], buf_d.at[j], sem_g))
        for dsc in pend:
            dsc.wait()
        pend = []
        for j in range(GRP):
            pend.append(pltpu.async_copy(
                buf_s.at[j], os_hbm.at[pl.ds(base0 + j * CB, CB)], sem_w))
            pend.append(pltpu.async_copy(
                buf_d.at[j], od_hbm.at[pl.ds(base0 + j * CB, CB)], sem_w))
        for dsc in pend:
            dsc.wait()
        return 0

    lax.fori_loop(0, CPW_SL // GRP, body, 0)


@functools.cache
def _scatter_phase():
    mesh = plsc.VectorSubcoreMesh(core_axis_name="c", subcore_axis_name="s")
    return functools.partial(
        pl.kernel,
        out_type=jax.ShapeDtypeStruct((NC * ACC_ROWS, 32), jnp.float32),
        mesh=mesh,
        scratch_types=[
            pltpu.VMEM((GRP, CB), jnp.int32),
            pltpu.VMEM((GRP, CB, 32), jnp.float32),
            pltpu.VMEM_SHARED((ACC_ROWS, 32), jnp.float32),
            pltpu.SemaphoreType.DMA,
            pltpu.SemaphoreType.DMA,
        ],
        compiler_params=pltpu.CompilerParams(use_tc_tiling_on_sc=False),
    )(_scatter_body)


def _scatter_body(dst_hbm, fe_hbm, init_hbm, out_hbm, idx_v, buf, acc,
                  sem_f, sem_s):
    cid = lax.axis_index("c")
    sid = lax.axis_index("s")

    pltpu.sync_copy(init_hbm.at[pl.ds(cid * ACC_ROWS + sid * RPT, RPT)],
                    acc.at[pl.ds(sid * RPT, RPT)])
    plsc.subcore_barrier()

    half = E_SL // NC

    def body(g, _):
        base0 = cid * half + (sid * CPT_SL + g * GRP) * CB
        pend = []
        for j in range(GRP):
            pend.append(pltpu.async_copy(
                dst_hbm.at[pl.ds(base0 + j * CB, CB)], idx_v.at[j], sem_f))
            pend.append(pltpu.async_copy(
                fe_hbm.at[pl.ds(base0 + j * CB, CB)], buf.at[j], sem_f))
        for dsc in pend:
            dsc.wait()
        pend = []
        for j in range(GRP):
            pend.append(pltpu.async_copy(
                buf.at[j], acc.at[idx_v.at[j]], sem_s, add=True))
        for dsc in pend:
            dsc.wait()
        return 0

    lax.fori_loop(0, CPT_SL // GRP, body, 0)

    plsc.subcore_barrier()
    pltpu.sync_copy(acc.at[pl.ds(sid * RPT, RPT)],
                    out_hbm.at[pl.ds(cid * ACC_ROWS + sid * RPT, RPT)])


def _dense_body(xs_ref, xd_ref, w1t_ref, w2t_ref, w3t_ref, out_ref):
    BE4 = BE // 4
    y = xs_ref[...].T
    xt = jnp.concatenate([y[32 * r:32 * r + 32] for r in range(4)], axis=1)
    z = xd_ref[...].T
    pdt = jnp.concatenate([z[32 * r:32 * r + 3] for r in range(4)], axis=1)
    vec = pdt - xt[0:3]
    r2 = vec[0:1] * vec[0:1] + vec[1:2] * vec[1:2] + vec[2:3] * vec[2:3] + 1e-12
    rinv = lax.rsqrt(r2)
    r = r2 * rinv
    y1 = _SQ3 * vec * rinv

    vals = _STEP * (1.0 + lax.broadcasted_iota(
        jnp.int32, (NBASIS, 1), 0).astype(jnp.float32))
    d = (r - vals) * (1.0 / _STEP)
    u = 1.0 - d * d
    good = u > 0.0
    emb = jnp.where(good, _EMBC * jnp.exp(-2.0 / jnp.where(good, u, 1.0)), 0.0)

    f32 = jnp.float32
    h = jnp.dot(w1t_ref[...], emb.astype(jnp.bfloat16),
                preferred_element_type=f32)
    h = (jnp.maximum(h, 0.0) * _SQ2).astype(jnp.bfloat16)
    h = jnp.dot(w2t_ref[...], h, preferred_element_type=f32)
    h = (jnp.maximum(h, 0.0) * _SQ2).astype(jnp.bfloat16)
    w = jnp.dot(w3t_ref[...], h, preferred_element_type=f32)

    s = xt[3:11]
    v = xt[11:23]

    dots = []
    for k in range(4):
        dk = (v[3 * k:3 * k + 1] * y1[0:1]
              + v[3 * k + 1:3 * k + 2] * y1[1:2]
              + v[3 * k + 2:3 * k + 3] * y1[2:3]) * (1.0 / _SQ3)
        dots.append(dk)

    out0 = s[0:1] * w[0:8]
    for i in range(1, 8):
        out0 = out0 + s[i:i + 1] * w[8 * i:8 * i + 8]
    for k in range(4):
        out0 = out0 + dots[k] * w[64 + 8 * k:72 + 8 * k]
    out0 = out0 * _ALPHA

    p = s[0:1] * w[96:100]
    for i in range(1, 8):
        p = p + s[i:i + 1] * w[96 + 4 * i:100 + 4 * i]
    q = []
    for c in range(3):
        qc = v[c:c + 1] * w[128:132]
        for k in range(1, 4):
            qc = qc + v[3 * k + c:3 * k + c + 1] * w[128 + 4 * k:132 + 4 * k]
        q.append(qc)

    rows = [out0]
    for c in range(3):
        rows.append((p * y1[c:c + 1] + q[c]) * _ALPHA)
    rows.append(jnp.zeros((12, BE), jnp.float32))
    fe = jnp.concatenate(rows, axis=0)
    ft = fe.T
    out_ref[...] = jnp.concatenate(
        [ft[r * BE4:(r + 1) * BE4] for r in range(4)], axis=1)


def _dense_phase(xs, xd, w1t, w2t, w3t):
    grid = (E_SL // BE,)
    return pl.pallas_call(
        _dense_body,
        grid=grid,
        in_specs=[
            pl.BlockSpec((BE // 4, 128), lambda i: (i, 0)),
            pl.BlockSpec((BE // 4, 128), lambda i: (i, 0)),
            pl.BlockSpec((20, 20), lambda i: (0, 0)),
            pl.BlockSpec((20, 20), lambda i: (0, 0)),
            pl.BlockSpec((144, 20), lambda i: (0, 0)),
        ],
        out_specs=pl.BlockSpec((BE // 4, 128), lambda i: (i, 0)),
        out_shape=jax.ShapeDtypeStruct((E_SL // 4, 128), jnp.float32),
    )(xs, xd, w1t, w2t, w3t)


BN_BLK = 3128
BN_NBLK = ACC_ROWS // BN_BLK


def _stats_body(*args):
    part_refs, (out_ref, acc_ref) = args[:-2], args[-2:]
    i = pl.program_id(0)

    @pl.when(i == 0)
    def _():
        acc_ref[...] = jnp.zeros_like(acc_ref)

    f = sum(ref[...] for ref in part_refs)
    rows = i * BN_BLK + lax.broadcasted_iota(jnp.int32, (BN_BLK, 32), 0)
    fm = jnp.where(rows < N_NODES, f, 0.0)
    acc_ref[0:1] += jnp.sum(fm, axis=0, keepdims=True)
    acc_ref[1:2] += jnp.sum(fm * fm, axis=0, keepdims=True)

    @pl.when(i == BN_NBLK - 1)
    def _():
        out_ref[...] = acc_ref[...]


def _norm_body(*args):
    part_refs = args[:-4]
    st_ref, grow_ref, brow_ref, out_ref = args[-4:]
    f = sum(ref[...] for ref in part_refs)
    inv_n = 1.0 / float(N_NODES)
    mu = st_ref[0:1] * inv_n
    sq = st_ref[1:2] * inv_n
    var = sq - mu * mu
    lane = lax.broadcasted_iota(jnp.int32, (32, 32), 0)
    lane_t = lax.broadcasted_iota(jnp.int32, (32, 32), 1)
    vlane = (lane >= 8) & (lane < 20) & (lane_t >= 8) & (lane_t < 20)
    m3 = jnp.where(vlane & ((lane - 8) % 4 == (lane_t - 8) % 4), 1.0, 0.0)
    n2 = sq @ m3
    s_lane = lax.broadcasted_iota(jnp.int32, (1, 32), 1) < 8
    denom = jnp.sqrt(jnp.where(s_lane, var, n2) + 1e-5)
    norm = jnp.where(s_lane, f - mu, f) / denom
    res = norm * grow_ref[...] + brow_ref[...]
    sblock = (lane == lane_t) & (lane_t < 8)
    vperm = vlane & (lane - 8 == ((lane_t - 8) % 3) * 4 + (lane_t - 8) // 3)
    pmat = jnp.where(sblock | vperm, 1.0, 0.0)
    res = res @ pmat
    out_ref[...] = res[:, 0:20]


def _bn_phase(parts, grow, brow):
    pa_spec = pl.BlockSpec((BN_BLK, 32), lambda i: (i, 0))
    pb_spec = pl.BlockSpec((BN_BLK, 32), lambda i: (i + BN_NBLK, 0))
    part_specs = [s for _ in parts for s in (pa_spec, pb_spec)]
    part_args = [x for pt in parts for x in (pt, pt)]
    stats = pl.pallas_call(
        _stats_body,
        grid=(BN_NBLK,),
        in_specs=part_specs,
        out_specs=pl.BlockSpec((2, 32), lambda i: (0, 0)),
        out_shape=jax.ShapeDtypeStruct((2, 32), jnp.float32),
        scratch_shapes=[pltpu.VMEM((2, 32), jnp.float32)],
    )(*part_args)
    return pl.pallas_call(
        _norm_body,
        grid=(BN_NBLK,),
        in_specs=part_specs + [
            pl.BlockSpec((2, 32), lambda i: (0, 0)),
            pl.BlockSpec((1, 32), lambda i: (0, 0)),
            pl.BlockSpec((1, 32), lambda i: (0, 0)),
        ],
        out_specs=pl.BlockSpec((BN_BLK, 20), lambda i: (i, 0)),
        out_shape=jax.ShapeDtypeStruct((N_NODES, 20), jnp.float32),
    )(*part_args, stats, grow, brow)


def kernel(pos, batch, f_in, edge_index, W1, W2, W3, gamma_s, beta_s, gamma_v):
    src = edge_index[0]
    dst = edge_index[1]
    pad = E_PAD - N_EDGES
    src_p = jnp.concatenate([src, jnp.zeros((pad,), jnp.int32)])
    dst_p = jnp.concatenate([dst, jnp.full((pad,), DUMP_ROW, jnp.int32)])

    tab32 = jnp.concatenate(
        [pos, f_in, jnp.zeros((N_NODES, 9), jnp.float32)], axis=1)

    w1t = (W1 * (1.0 / np.sqrt(float(NBASIS)))).T.astype(jnp.bfloat16)
    w2t = (W2 * (1.0 / np.sqrt(20.0))).T.astype(jnp.bfloat16)
    w3t = (W3 * (1.0 / np.sqrt(20.0))).T.astype(jnp.bfloat16)
    zeros_acc = jnp.zeros((NC * ACC_ROWS, 32), jnp.float32)

    srcs = [lax.slice(src_p, (k * E_SL,), ((k + 1) * E_SL,))
            for k in range(NSLICE)]
    dsts = [lax.slice(dst_p, (k * E_SL,), ((k + 1) * E_SL,))
            for k in range(NSLICE)]
    gs = [_gather_phase()(srcs[k], dsts[k], tab32) for k in range(NSLICE)]
    running = zeros_acc
    for k in range(NSLICE):
        xs, xd = gs[k]
        xs = xs.reshape(E_SL // 4, 128)
        xd = xd.reshape(E_SL // 4, 128)
        fe = _dense_phase(xs, xd, w1t, w2t, w3t).reshape(E_SL, 32)
        running = _scatter_phase()(dsts[k], fe, running)
    parts = [running]

    grow = jnp.concatenate(
        [gamma_s, jnp.tile(gamma_v, 3), jnp.zeros((12,), jnp.float32)]
    ).reshape(1, 32)
    brow = jnp.concatenate(
        [beta_s, jnp.zeros((24,), jnp.float32)]).reshape(1, 32)
    return _bn_phase(parts, grow, brow)


# speedup vs baseline: 21.2103x; 1.1670x over previous
import functools

import jax
import jax.numpy as jnp
import numpy as np
from jax import lax
from jax.experimental import pallas as pl
from jax.experimental.pallas import tpu as pltpu
from jax.experimental.pallas import tpu_sc as plsc

N_NODES = 50000
N_EDGES = 800000
RADIUS = 5.0
NBASIS = 20

NC, NS = 2, 16
NW = NC * NS
CB = 128
CPW = 196
E_PAD = NW * CPW * CB
NSLICE = 4
GRP = 7
E_SL = E_PAD // NSLICE
CPW_SL = CPW // NSLICE
CPT_SL = E_SL // NC // NS // CB
ACC_ROWS = 50048
RPT = ACC_ROWS // NS
DUMP_ROW = N_NODES

BE = 4096

_SQ2 = float(np.sqrt(2.0))
_SQ3 = float(np.sqrt(3.0))
_ALPHA = float(1.0 / np.sqrt(12.0))
_EMBC = float(1.14136 * np.exp(2.0) * np.sqrt(float(NBASIS)))
_STEP = float(RADIUS / (NBASIS + 1))

@functools.cache
def _gather_phase(slice_k):
    mesh = plsc.VectorSubcoreMesh(core_axis_name="c", subcore_axis_name="s")
    return functools.partial(
        pl.kernel,
        out_type=(
            jax.ShapeDtypeStruct((E_SL, 32), jnp.float32),
            jax.ShapeDtypeStruct((E_SL, 32), jnp.float32),
        ),
        mesh=mesh,
        scratch_types=[
            pltpu.VMEM((GRP, CB), jnp.int32),
            pltpu.VMEM((GRP, CB), jnp.int32),
            pltpu.VMEM((GRP, CB, 32), jnp.float32),
            pltpu.VMEM((GRP, CB, 32), jnp.float32),
            pltpu.SemaphoreType.DMA,
            pltpu.SemaphoreType.DMA,
            pltpu.SemaphoreType.DMA,
        ],
        compiler_params=pltpu.CompilerParams(use_tc_tiling_on_sc=False),
    )(functools.partial(_gather_body, slice_k))


def _gather_body(slice_k, src_hbm, dst_hbm, tab32_hbm, os_hbm, od_hbm,
                 idx_s, idx_d, buf_s, buf_d, sem_i, sem_g, sem_w):
    wid = lax.axis_index("s") * NC + lax.axis_index("c")

    def body(g, _):
        base0 = (wid * CPW_SL + g * GRP) * CB
        ibase0 = slice_k * E_SL + base0
        pend = []
        for j in range(GRP):
            pend.append(pltpu.async_copy(
                src_hbm.at[pl.ds(ibase0 + j * CB, CB)], idx_s.at[j], sem_i))
            pend.append(pltpu.async_copy(
                dst_hbm.at[pl.ds(ibase0 + j * CB, CB)], idx_d.at[j], sem_i))
        for dsc in pend:
            dsc.wait()
        pend = []
        for j in range(GRP):
            pend.append(pltpu.async_copy(
                tab32_hbm.at[idx_s.at[j]], buf_s.at[j], sem_g))
            pend.append(pltpu.async_copy(
                tab32_hbm.at[idx_d.at[j]], buf_d.at[j], sem_g))
        for dsc in pend:
            dsc.wait()
        pend = []
        for j in range(GRP):
            pend.append(pltpu.async_copy(
                buf_s.at[j], os_hbm.at[pl.ds(base0 + j * CB, CB)], sem_w))
            pend.append(pltpu.async_copy(
                buf_d.at[j], od_hbm.at[pl.ds(base0 + j * CB, CB)], sem_w))
        for dsc in pend:
            dsc.wait()
        return 0

    lax.fori_loop(0, CPW_SL // GRP, body, 0)


@functools.cache
def _scatter_phase(slice_k):
    mesh = plsc.VectorSubcoreMesh(core_axis_name="c", subcore_axis_name="s")
    return functools.partial(
        pl.kernel,
        out_type=jax.ShapeDtypeStruct((NC * ACC_ROWS, 32), jnp.float32),
        mesh=mesh,
        scratch_types=[
            pltpu.VMEM((GRP, CB), jnp.int32),
            pltpu.VMEM((GRP, CB, 32), jnp.float32),
            pltpu.VMEM_SHARED((ACC_ROWS, 32), jnp.float32),
            pltpu.SemaphoreType.DMA,
            pltpu.SemaphoreType.DMA,
        ],
        compiler_params=pltpu.CompilerParams(use_tc_tiling_on_sc=False),
    )(functools.partial(_scatter_body, slice_k))


def _scatter_body(slice_k, dst_hbm, fe_hbm, init_hbm, out_hbm, idx_v, buf,
                  acc, sem_f, sem_s):
    cid = lax.axis_index("c")
    sid = lax.axis_index("s")

    pltpu.sync_copy(init_hbm.at[pl.ds(cid * ACC_ROWS + sid * RPT, RPT)],
                    acc.at[pl.ds(sid * RPT, RPT)])
    plsc.subcore_barrier()

    half = E_SL // NC

    def body(g, _):
        base0 = cid * half + (sid * CPT_SL + g * GRP) * CB
        ibase0 = slice_k * E_SL + base0
        pend = []
        for j in range(GRP):
            pend.append(pltpu.async_copy(
                dst_hbm.at[pl.ds(ibase0 + j * CB, CB)], idx_v.at[j], sem_f))
            pend.append(pltpu.async_copy(
                fe_hbm.at[pl.ds(base0 + j * CB, CB)], buf.at[j], sem_f))
        for dsc in pend:
            dsc.wait()
        pend = []
        for j in range(GRP):
            pend.append(pltpu.async_copy(
                buf.at[j], acc.at[idx_v.at[j]], sem_s, add=True))
        for dsc in pend:
            dsc.wait()
        return 0

    lax.fori_loop(0, CPT_SL // GRP, body, 0)

    plsc.subcore_barrier()
    pltpu.sync_copy(acc.at[pl.ds(sid * RPT, RPT)],
                    out_hbm.at[pl.ds(cid * ACC_ROWS + sid * RPT, RPT)])


def _dense_body(xs_ref, xd_ref, w1t_ref, w2t_ref, w3t_ref, out_ref):
    BE4 = BE // 4
    y = xs_ref[...].T
    xt = jnp.concatenate([y[32 * r:32 * r + 32] for r in range(4)], axis=1)
    z = xd_ref[...].T
    pdt = jnp.concatenate([z[32 * r:32 * r + 3] for r in range(4)], axis=1)
    vec = pdt - xt[0:3]
    r2 = vec[0:1] * vec[0:1] + vec[1:2] * vec[1:2] + vec[2:3] * vec[2:3] + 1e-12
    rinv = lax.rsqrt(r2)
    r = r2 * rinv
    y1 = _SQ3 * vec * rinv

    vals = _STEP * (1.0 + lax.broadcasted_iota(
        jnp.int32, (NBASIS, 1), 0).astype(jnp.float32))
    d = (r - vals) * (1.0 / _STEP)
    u = 1.0 - d * d
    good = u > 0.0
    emb = jnp.where(good, _EMBC * jnp.exp(-2.0 / jnp.where(good, u, 1.0)), 0.0)

    f32 = jnp.float32
    h = jnp.dot(w1t_ref[...], emb.astype(jnp.bfloat16),
                preferred_element_type=f32)
    h = (jnp.maximum(h, 0.0) * _SQ2).astype(jnp.bfloat16)
    h = jnp.dot(w2t_ref[...], h, preferred_element_type=f32)
    h = (jnp.maximum(h, 0.0) * _SQ2).astype(jnp.bfloat16)
    w = jnp.dot(w3t_ref[...], h, preferred_element_type=f32)

    s = xt[3:11]
    v = xt[11:23]

    dots = []
    for k in range(4):
        dk = (v[3 * k:3 * k + 1] * y1[0:1]
              + v[3 * k + 1:3 * k + 2] * y1[1:2]
              + v[3 * k + 2:3 * k + 3] * y1[2:3]) * (1.0 / _SQ3)
        dots.append(dk)

    out0 = s[0:1] * w[0:8]
    for i in range(1, 8):
        out0 = out0 + s[i:i + 1] * w[8 * i:8 * i + 8]
    for k in range(4):
        out0 = out0 + dots[k] * w[64 + 8 * k:72 + 8 * k]
    out0 = out0 * _ALPHA

    p = s[0:1] * w[96:100]
    for i in range(1, 8):
        p = p + s[i:i + 1] * w[96 + 4 * i:100 + 4 * i]
    q = []
    for c in range(3):
        qc = v[c:c + 1] * w[128:132]
        for k in range(1, 4):
            qc = qc + v[3 * k + c:3 * k + c + 1] * w[128 + 4 * k:132 + 4 * k]
        q.append(qc)

    rows = [out0]
    for c in range(3):
        rows.append((p * y1[c:c + 1] + q[c]) * _ALPHA)
    rows.append(jnp.zeros((12, BE), jnp.float32))
    fe = jnp.concatenate(rows, axis=0)
    ft = fe.T
    out_ref[...] = jnp.concatenate(
        [ft[r * BE4:(r + 1) * BE4] for r in range(4)], axis=1)


def _dense_phase(xs, xd, w1t, w2t, w3t):
    grid = (E_SL // BE,)
    return pl.pallas_call(
        _dense_body,
        grid=grid,
        in_specs=[
            pl.BlockSpec((BE // 4, 128), lambda i: (i, 0)),
            pl.BlockSpec((BE // 4, 128), lambda i: (i, 0)),
            pl.BlockSpec((20, 20), lambda i: (0, 0)),
            pl.BlockSpec((20, 20), lambda i: (0, 0)),
            pl.BlockSpec((144, 20), lambda i: (0, 0)),
        ],
        out_specs=pl.BlockSpec((BE // 4, 128), lambda i: (i, 0)),
        out_shape=jax.ShapeDtypeStruct((E_SL // 4, 128), jnp.float32),
    )(xs, xd, w1t, w2t, w3t)


BN_BLK = 3128
BN_NBLK = ACC_ROWS // BN_BLK


def _stats_body(*args):
    part_refs, (out_ref, acc_ref) = args[:-2], args[-2:]
    i = pl.program_id(0)

    @pl.when(i == 0)
    def _():
        acc_ref[...] = jnp.zeros_like(acc_ref)

    f = sum(ref[...] for ref in part_refs)
    rows = i * BN_BLK + lax.broadcasted_iota(jnp.int32, (BN_BLK, 32), 0)
    fm = jnp.where(rows < N_NODES, f, 0.0)
    acc_ref[0:1] += jnp.sum(fm, axis=0, keepdims=True)
    acc_ref[1:2] += jnp.sum(fm * fm, axis=0, keepdims=True)

    @pl.when(i == BN_NBLK - 1)
    def _():
        out_ref[...] = acc_ref[...]


def _norm_body(*args):
    part_refs = args[:-4]
    st_ref, grow_ref, brow_ref, out_ref = args[-4:]
    f = sum(ref[...] for ref in part_refs)
    inv_n = 1.0 / float(N_NODES)
    mu = st_ref[0:1] * inv_n
    sq = st_ref[1:2] * inv_n
    var = sq - mu * mu
    lane = lax.broadcasted_iota(jnp.int32, (32, 32), 0)
    lane_t = lax.broadcasted_iota(jnp.int32, (32, 32), 1)
    vlane = (lane >= 8) & (lane < 20) & (lane_t >= 8) & (lane_t < 20)
    m3 = jnp.where(vlane & ((lane - 8) % 4 == (lane_t - 8) % 4), 1.0, 0.0)
    n2 = sq @ m3
    s_lane = lax.broadcasted_iota(jnp.int32, (1, 32), 1) < 8
    denom = jnp.sqrt(jnp.where(s_lane, var, n2) + 1e-5)
    norm = jnp.where(s_lane, f - mu, f) / denom
    res = norm * grow_ref[...] + brow_ref[...]
    sblock = (lane == lane_t) & (lane_t < 8)
    vperm = vlane & (lane - 8 == ((lane_t - 8) % 3) * 4 + (lane_t - 8) // 3)
    pmat = jnp.where(sblock | vperm, 1.0, 0.0)
    res = res @ pmat
    out_ref[...] = res[:, 0:20]


def _bn_phase(parts, grow, brow):
    pa_spec = pl.BlockSpec((BN_BLK, 32), lambda i: (i, 0))
    pb_spec = pl.BlockSpec((BN_BLK, 32), lambda i: (i + BN_NBLK, 0))
    part_specs = [s for _ in parts for s in (pa_spec, pb_spec)]
    part_args = [x for pt in parts for x in (pt, pt)]
    stats = pl.pallas_call(
        _stats_body,
        grid=(BN_NBLK,),
        in_specs=part_specs,
        out_specs=pl.BlockSpec((2, 32), lambda i: (0, 0)),
        out_shape=jax.ShapeDtypeStruct((2, 32), jnp.float32),
        scratch_shapes=[pltpu.VMEM((2, 32), jnp.float32)],
    )(*part_args)
    return pl.pallas_call(
        _norm_body,
        grid=(BN_NBLK,),
        in_specs=part_specs + [
            pl.BlockSpec((2, 32), lambda i: (0, 0)),
            pl.BlockSpec((1, 32), lambda i: (0, 0)),
            pl.BlockSpec((1, 32), lambda i: (0, 0)),
        ],
        out_specs=pl.BlockSpec((BN_BLK, 20), lambda i: (i, 0)),
        out_shape=jax.ShapeDtypeStruct((N_NODES, 20), jnp.float32),
    )(*part_args, stats, grow, brow)


def kernel(pos, batch, f_in, edge_index, W1, W2, W3, gamma_s, beta_s, gamma_v):
    src = edge_index[0]
    dst = edge_index[1]
    pad = E_PAD - N_EDGES
    src_p = jnp.concatenate([src, jnp.zeros((pad,), jnp.int32)])
    dst_p = jnp.concatenate([dst, jnp.full((pad,), DUMP_ROW, jnp.int32)])

    tab32 = jnp.concatenate(
        [pos, f_in, jnp.zeros((N_NODES, 9), jnp.float32)], axis=1)

    w1t = (W1 * (1.0 / np.sqrt(float(NBASIS)))).T.astype(jnp.bfloat16)
    w2t = (W2 * (1.0 / np.sqrt(20.0))).T.astype(jnp.bfloat16)
    w3t = (W3 * (1.0 / np.sqrt(20.0))).T.astype(jnp.bfloat16)
    zeros_acc = jnp.zeros((NC * ACC_ROWS, 32), jnp.float32)

    gs = [_gather_phase(k)(src_p, dst_p, tab32) for k in range(NSLICE)]
    running = zeros_acc
    for k in range(NSLICE):
        xs, xd = gs[k]
        xs = xs.reshape(E_SL // 4, 128)
        xd = xd.reshape(E_SL // 4, 128)
        fe = _dense_phase(xs, xd, w1t, w2t, w3t).reshape(E_SL, 32)
        running = _scatter_phase(k)(dst_p, fe, running)
    parts = [running]

    grow = jnp.concatenate(
        [gamma_s, jnp.tile(gamma_v, 3), jnp.zeros((12,), jnp.float32)]
    ).reshape(1, 32)
    brow = jnp.concatenate(
        [beta_s, jnp.zeros((24,), jnp.float32)]).reshape(1, 32)
    return _bn_phase(parts, grow, brow)


# speedup vs baseline: 22.0560x; 1.0399x over previous
import functools

import jax
import jax.numpy as jnp
import numpy as np
from jax import lax
from jax.experimental import pallas as pl
from jax.experimental.pallas import tpu as pltpu
from jax.experimental.pallas import tpu_sc as plsc

N_NODES = 50000
N_EDGES = 800000
RADIUS = 5.0
NBASIS = 20

NC, NS = 2, 16
NW = NC * NS
CB = 128
CPW = 196
E_PAD = NW * CPW * CB
NSLICE = 4
GRP = 7
E_SL = E_PAD // NSLICE
CPW_SL = CPW // NSLICE
CPT_SL = E_SL // NC // NS // CB
ACC_ROWS = 50048
RPT = ACC_ROWS // NS
DUMP_ROW = N_NODES

BE = 7168

_SQ2 = float(np.sqrt(2.0))
_SQ3 = float(np.sqrt(3.0))
_ALPHA = float(1.0 / np.sqrt(12.0))
_EMBC = float(1.14136 * np.exp(2.0) * np.sqrt(float(NBASIS)))
_STEP = float(RADIUS / (NBASIS + 1))

@functools.cache
def _gather_phase(slice_k):
    mesh = plsc.VectorSubcoreMesh(core_axis_name="c", subcore_axis_name="s")
    return functools.partial(
        pl.kernel,
        out_type=(
            jax.ShapeDtypeStruct((E_SL, 32), jnp.float32),
            jax.ShapeDtypeStruct((E_SL, 32), jnp.float32),
        ),
        mesh=mesh,
        scratch_types=[
            pltpu.VMEM((GRP, CB), jnp.int32),
            pltpu.VMEM((GRP, CB), jnp.int32),
            pltpu.VMEM((GRP, CB, 32), jnp.float32),
            pltpu.VMEM((GRP, CB, 32), jnp.float32),
            pltpu.SemaphoreType.DMA,
            pltpu.SemaphoreType.DMA,
            pltpu.SemaphoreType.DMA,
        ],
        compiler_params=pltpu.CompilerParams(use_tc_tiling_on_sc=False),
    )(functools.partial(_gather_body, slice_k))


def _gather_body(slice_k, src_hbm, dst_hbm, tab32_hbm, os_hbm, od_hbm,
                 idx_s, idx_d, buf_s, buf_d, sem_i, sem_g, sem_w):
    wid = lax.axis_index("s") * NC + lax.axis_index("c")

    def body(g, _):
        base0 = (wid * CPW_SL + g * GRP) * CB
        ibase0 = slice_k * E_SL + base0
        pend = []
        for j in range(GRP):
            pend.append(pltpu.async_copy(
                src_hbm.at[pl.ds(ibase0 + j * CB, CB)], idx_s.at[j], sem_i))
            pend.append(pltpu.async_copy(
                dst_hbm.at[pl.ds(ibase0 + j * CB, CB)], idx_d.at[j], sem_i))
        for dsc in pend:
            dsc.wait()
        pend = []
        for j in range(GRP):
            pend.append(pltpu.async_copy(
                tab32_hbm.at[idx_s.at[j]], buf_s.at[j], sem_g))
            pend.append(pltpu.async_copy(
                tab32_hbm.at[idx_d.at[j]], buf_d.at[j], sem_g))
        for dsc in pend:
            dsc.wait()
        pend = []
        for j in range(GRP):
            pend.append(pltpu.async_copy(
                buf_s.at[j], os_hbm.at[pl.ds(base0 + j * CB, CB)], sem_w))
            pend.append(pltpu.async_copy(
                buf_d.at[j], od_hbm.at[pl.ds(base0 + j * CB, CB)], sem_w))
        for dsc in pend:
            dsc.wait()
        return 0

    lax.fori_loop(0, CPW_SL // GRP, body, 0)


@functools.cache
def _scatter_phase(slice_k):
    mesh = plsc.VectorSubcoreMesh(core_axis_name="c", subcore_axis_name="s")
    return functools.partial(
        pl.kernel,
        out_type=jax.ShapeDtypeStruct((NC * ACC_ROWS, 32), jnp.float32),
        mesh=mesh,
        scratch_types=[
            pltpu.VMEM((GRP, CB), jnp.int32),
            pltpu.VMEM((GRP, CB, 32), jnp.float32),
            pltpu.VMEM_SHARED((ACC_ROWS, 32), jnp.float32),
            pltpu.SemaphoreType.DMA,
            pltpu.SemaphoreType.DMA,
        ],
        compiler_params=pltpu.CompilerParams(use_tc_tiling_on_sc=False),
    )(functools.partial(_scatter_body, slice_k))


def _scatter_body(slice_k, dst_hbm, fe_hbm, init_hbm, out_hbm, idx_v, buf,
                  acc, sem_f, sem_s):
    cid = lax.axis_index("c")
    sid = lax.axis_index("s")

    pltpu.sync_copy(init_hbm.at[pl.ds(cid * ACC_ROWS + sid * RPT, RPT)],
                    acc.at[pl.ds(sid * RPT, RPT)])
    plsc.subcore_barrier()

    half = E_SL // NC

    def body(g, _):
        base0 = cid * half + (sid * CPT_SL + g * GRP) * CB
        ibase0 = slice_k * E_SL + base0
        pend = []
        for j in range(GRP):
            pend.append(pltpu.async_copy(
                dst_hbm.at[pl.ds(ibase0 + j * CB, CB)], idx_v.at[j], sem_f))
            pend.append(pltpu.async_copy(
                fe_hbm.at[pl.ds(base0 + j * CB, CB)], buf.at[j], sem_f))
        for dsc in pend:
            dsc.wait()
        pend = []
        for j in range(GRP):
            pend.append(pltpu.async_copy(
                buf.at[j], acc.at[idx_v.at[j]], sem_s, add=True))
        for dsc in pend:
            dsc.wait()
        return 0

    lax.fori_loop(0, CPT_SL // GRP, body, 0)

    plsc.subcore_barrier()
    pltpu.sync_copy(acc.at[pl.ds(sid * RPT, RPT)],
                    out_hbm.at[pl.ds(cid * ACC_ROWS + sid * RPT, RPT)])


def _dense_body(xs_ref, xd_ref, w1t_ref, w2t_ref, w3t_ref, out_ref):
    BE4 = BE // 4
    y = xs_ref[...].T
    xt = jnp.concatenate([y[32 * r:32 * r + 32] for r in range(4)], axis=1)
    z = xd_ref[...].T
    pdt = jnp.concatenate([z[32 * r:32 * r + 3] for r in range(4)], axis=1)
    vec = pdt - xt[0:3]
    r2 = vec[0:1] * vec[0:1] + vec[1:2] * vec[1:2] + vec[2:3] * vec[2:3] + 1e-12
    rinv = lax.rsqrt(r2)
    r = r2 * rinv
    y1 = _SQ3 * vec * rinv

    vals = _STEP * (1.0 + lax.broadcasted_iota(
        jnp.int32, (NBASIS, 1), 0).astype(jnp.float32))
    d = (r - vals) * (1.0 / _STEP)
    u = 1.0 - d * d
    good = u > 0.0
    emb = jnp.where(good, _EMBC * jnp.exp(-2.0 / jnp.where(good, u, 1.0)), 0.0)

    f32 = jnp.float32
    h = jnp.dot(w1t_ref[...], emb.astype(jnp.bfloat16),
                preferred_element_type=f32)
    h = (jnp.maximum(h, 0.0) * _SQ2).astype(jnp.bfloat16)
    h = jnp.dot(w2t_ref[...], h, preferred_element_type=f32)
    h = (jnp.maximum(h, 0.0) * _SQ2).astype(jnp.bfloat16)
    w = jnp.dot(w3t_ref[...], h, preferred_element_type=f32)

    s = xt[3:11]
    v = xt[11:23]

    dots = []
    for k in range(4):
        dk = (v[3 * k:3 * k + 1] * y1[0:1]
              + v[3 * k + 1:3 * k + 2] * y1[1:2]
              + v[3 * k + 2:3 * k + 3] * y1[2:3]) * (1.0 / _SQ3)
        dots.append(dk)

    out0 = s[0:1] * w[0:8]
    for i in range(1, 8):
        out0 = out0 + s[i:i + 1] * w[8 * i:8 * i + 8]
    for k in range(4):
        out0 = out0 + dots[k] * w[64 + 8 * k:72 + 8 * k]
    out0 = out0 * _ALPHA

    p = s[0:1] * w[96:100]
    for i in range(1, 8):
        p = p + s[i:i + 1] * w[96 + 4 * i:100 + 4 * i]
    q = []
    for c in range(3):
        qc = v[c:c + 1] * w[128:132]
        for k in range(1, 4):
            qc = qc + v[3 * k + c:3 * k + c + 1] * w[128 + 4 * k:132 + 4 * k]
        q.append(qc)

    rows = [out0]
    for c in range(3):
        rows.append((p * y1[c:c + 1] + q[c]) * _ALPHA)
    rows.append(jnp.zeros((12, BE), jnp.float32))
    fe = jnp.concatenate(rows, axis=0)
    ft = fe.T
    out_ref[...] = jnp.concatenate(
        [ft[r * BE4:(r + 1) * BE4] for r in range(4)], axis=1)


def _dense_phase(xs, xd, w1t, w2t, w3t):
    grid = (E_SL // BE,)
    return pl.pallas_call(
        _dense_body,
        grid=grid,
        in_specs=[
            pl.BlockSpec((BE // 4, 128), lambda i: (i, 0)),
            pl.BlockSpec((BE // 4, 128), lambda i: (i, 0)),
            pl.BlockSpec((20, 20), lambda i: (0, 0)),
            pl.BlockSpec((20, 20), lambda i: (0, 0)),
            pl.BlockSpec((144, 20), lambda i: (0, 0)),
        ],
        out_specs=pl.BlockSpec((BE // 4, 128), lambda i: (i, 0)),
        out_shape=jax.ShapeDtypeStruct((E_SL // 4, 128), jnp.float32),
    )(xs, xd, w1t, w2t, w3t)


BN_BLK = 3128
BN_NBLK = ACC_ROWS // BN_BLK


def _stats_body(*args):
    part_refs, (out_ref, acc_ref) = args[:-2], args[-2:]
    i = pl.program_id(0)

    @pl.when(i == 0)
    def _():
        acc_ref[...] = jnp.zeros_like(acc_ref)

    f = sum(ref[...] for ref in part_refs)
    rows = i * BN_BLK + lax.broadcasted_iota(jnp.int32, (BN_BLK, 32), 0)
    fm = jnp.where(rows < N_NODES, f, 0.0)
    acc_ref[0:1] += jnp.sum(fm, axis=0, keepdims=True)
    acc_ref[1:2] += jnp.sum(fm * fm, axis=0, keepdims=True)

    @pl.when(i == BN_NBLK - 1)
    def _():
        out_ref[...] = acc_ref[...]


def _norm_body(*args):
    part_refs = args[:-4]
    st_ref, grow_ref, brow_ref, out_ref = args[-4:]
    f = sum(ref[...] for ref in part_refs)
    inv_n = 1.0 / float(N_NODES)
    mu = st_ref[0:1] * inv_n
    sq = st_ref[1:2] * inv_n
    var = sq - mu * mu
    lane = lax.broadcasted_iota(jnp.int32, (32, 32), 0)
    lane_t = lax.broadcasted_iota(jnp.int32, (32, 32), 1)
    vlane = (lane >= 8) & (lane < 20) & (lane_t >= 8) & (lane_t < 20)
    m3 = jnp.where(vlane & ((lane - 8) % 4 == (lane_t - 8) % 4), 1.0, 0.0)
    n2 = sq @ m3
    s_lane = lax.broadcasted_iota(jnp.int32, (1, 32), 1) < 8
    denom = jnp.sqrt(jnp.where(s_lane, var, n2) + 1e-5)
    norm = jnp.where(s_lane, f - mu, f) / denom
    res = norm * grow_ref[...] + brow_ref[...]
    sblock = (lane == lane_t) & (lane_t < 8)
    vperm = vlane & (lane - 8 == ((lane_t - 8) % 3) * 4 + (lane_t - 8) // 3)
    pmat = jnp.where(sblock | vperm, 1.0, 0.0)
    res = res @ pmat
    out_ref[...] = res[:, 0:20]


def _bn_phase(parts, grow, brow):
    pa_spec = pl.BlockSpec((BN_BLK, 32), lambda i: (i, 0))
    pb_spec = pl.BlockSpec((BN_BLK, 32), lambda i: (i + BN_NBLK, 0))
    part_specs = [s for _ in parts for s in (pa_spec, pb_spec)]
    part_args = [x for pt in parts for x in (pt, pt)]
    stats = pl.pallas_call(
        _stats_body,
        grid=(BN_NBLK,),
        in_specs=part_specs,
        out_specs=pl.BlockSpec((2, 32), lambda i: (0, 0)),
        out_shape=jax.ShapeDtypeStruct((2, 32), jnp.float32),
        scratch_shapes=[pltpu.VMEM((2, 32), jnp.float32)],
    )(*part_args)
    return pl.pallas_call(
        _norm_body,
        grid=(BN_NBLK,),
        in_specs=part_specs + [
            pl.BlockSpec((2, 32), lambda i: (0, 0)),
            pl.BlockSpec((1, 32), lambda i: (0, 0)),
            pl.BlockSpec((1, 32), lambda i: (0, 0)),
        ],
        out_specs=pl.BlockSpec((BN_BLK, 20), lambda i: (i, 0)),
        out_shape=jax.ShapeDtypeStruct((N_NODES, 20), jnp.float32),
    )(*part_args, stats, grow, brow)


def kernel(pos, batch, f_in, edge_index, W1, W2, W3, gamma_s, beta_s, gamma_v):
    src = edge_index[0]
    dst = edge_index[1]
    pad = E_PAD - N_EDGES
    src_p = jnp.concatenate([src, jnp.zeros((pad,), jnp.int32)])
    dst_p = jnp.concatenate([dst, jnp.full((pad,), DUMP_ROW, jnp.int32)])

    tab32 = jnp.concatenate(
        [pos, f_in, jnp.zeros((N_NODES, 9), jnp.float32)], axis=1)

    w1t = (W1 * (1.0 / np.sqrt(float(NBASIS)))).T.astype(jnp.bfloat16)
    w2t = (W2 * (1.0 / np.sqrt(20.0))).T.astype(jnp.bfloat16)
    w3t = (W3 * (1.0 / np.sqrt(20.0))).T.astype(jnp.bfloat16)
    zeros_acc = jnp.zeros((NC * ACC_ROWS, 32), jnp.float32)

    gs = [_gather_phase(k)(src_p, dst_p, tab32) for k in range(NSLICE)]
    running = zeros_acc
    for k in range(NSLICE):
        xs, xd = gs[k]
        xs = xs.reshape(E_SL // 4, 128)
        xd = xd.reshape(E_SL // 4, 128)
        fe = _dense_phase(xs, xd, w1t, w2t, w3t).reshape(E_SL, 32)
        running = _scatter_phase(k)(dst_p, fe, running)
    parts = [running]

    grow = jnp.concatenate(
        [gamma_s, jnp.tile(gamma_v, 3), jnp.zeros((12,), jnp.float32)]
    ).reshape(1, 32)
    brow = jnp.concatenate(
        [beta_s, jnp.zeros((24,), jnp.float32)]).reshape(1, 32)
    return _bn_phase(parts, grow, brow)
